# baseline (speedup 1.0000x reference)
import functools

import jax
import jax.numpy as jnp
from jax import lax
from jax.experimental import pallas as pl
from jax.experimental.pallas import tpu as pltpu
from jax.experimental.pallas import tpu_sc as plsc

N, EH, NNZ, C, NCLS, NGRAPH, NLAYER, ALPHA = 10000, 5000, 320000, 128, 10, 16, 2, 0.5
EPS = 1e-5
NP = 10240
EP = 5120
NC, NS = 2, 16
NW = NC * NS
PAIRS_W = NNZ // NW
K = 80
NCHUNK = PAIRS_W // K
TV = NP // NS
TE = EP // NS
ROWB = 2000
GRID = N // ROWB
EROWB = 1000
EGRID = EH // EROWB

_SC_MESH = plsc.VectorSubcoreMesh(core_axis_name="c", subcore_axis_name="s")
_SC_PARAMS = pltpu.CompilerParams(needs_layout_passes=False)


def _ln(x, g, b):
    m = jnp.mean(x, axis=-1, keepdims=True)
    v = jnp.mean((x - m) ** 2, axis=-1, keepdims=True)
    return (x - m) * lax.rsqrt(v + EPS) * g + b


def _dot(a, b):
    return jnp.dot(a, b, preferred_element_type=jnp.float32)


def _k_input(x_ref, w_ref, b_ref, o_ref):
    o_ref[...] = jnp.maximum(_dot(x_ref[...], w_ref[...]) + b_ref[...], 0.0)


def _k_prep(gb_ref, w2_ref, b2_ref, o_ref):
    o = _dot(gb_ref[...], w2_ref[...])
    o_ref[...] = o + jnp.concatenate(
        [jnp.zeros_like(b2_ref[...]), b2_ref[...]], axis=0)


def _k_layer_a(x_ref, g0, b0, w10, b10, g1, b1, w11, b11, g2lo, w2top,
               h2_o, a_o, sv_o, qv_o):
    x = x_ref[...]
    h = _ln(x, g0[...], b0[...])
    h = jnp.maximum(_dot(h, w10[...]) + b10[...], 0.0)
    h = _ln(h, g1[...], b1[...])
    h2_o[...] = _dot(h, w11[...]) + b11[...]
    a_o[...] = _dot(x * g2lo[...], w2top[...])
    sv_o[...] = jnp.sum(x, axis=-1, keepdims=True) * (1.0 / (2 * C))
    qv_o[...] = jnp.sum(x * x, axis=-1, keepdims=True) * (1.0 / (2 * C))


def _k_edge(xs_ref, ce_ref, g2hi, w2bot, b_o, se_o, qe_o):
    xs = xs_ref[...]
    cnt = ce_ref[0] + ce_ref[1]
    xe = (xs[0] + xs[1]) / jnp.clip(cnt, 1.0, None)
    b_o[...] = _dot(xe * g2hi[...], w2bot[...])
    se_o[...] = jnp.sum(xe, axis=-1, keepdims=True) * (1.0 / (2 * C))
    qe_o[...] = jnp.sum(xe * xe, axis=-1, keepdims=True) * (1.0 / (2 * C))


def _k_vertex(sb_ref, r_ref, mr_ref, cv_ref, a_ref, x0_ref, gwc_ref,
              g3, b3, w3, b3l, x_o):
    cnt = cv_ref[0] + cv_ref[1]
    gw = gwc_ref[0:1]
    bwc = gwc_ref[1:2]
    seg = (a_ref[...] * (r_ref[0] + r_ref[1])
           + (sb_ref[0] + sb_ref[1])
           - (mr_ref[0] + mr_ref[1]) * gw
           + cnt * bwc)
    xv = seg / jnp.clip(cnt, 1.0, None)
    xn = (1.0 - ALPHA) * xv + ALPHA * x0_ref[...]
    x_o[...] = jnp.maximum(_dot(_ln(xn, g3[...], b3[...]), w3[...]) + b3l[...], 0.0)


def _k_clf(x_ref, wc0, bc0, gc, bc, wc1, bc1, ab_ref, out_ref, sum_scr, cnt_scr):
    i = pl.program_id(0)

    @pl.when(i == 0)
    def _():
        sum_scr[...] = jnp.zeros_like(sum_scr)
        cnt_scr[...] = jnp.zeros_like(cnt_scr)

    h = jnp.maximum(_dot(x_ref[...], wc0[...]) + bc0[...], 0.0)
    h = _ln(h, gc[...], bc[...])
    o = _dot(h, wc1[...]) + bc1[...]
    ab = ab_ref[0]
    gids = lax.broadcasted_iota(jnp.int32, (NGRAPH, ROWB), 0)
    oh = jnp.where(ab == gids, 1.0, 0.0)
    sum_scr[...] += _dot(oh, o)
    cnt_scr[...] += jnp.broadcast_to(
        jnp.sum(oh, axis=-1, keepdims=True), (NGRAPH, C))

    @pl.when(i == GRID - 1)
    def _():
        out_ref[...] = (sum_scr[...] / jnp.clip(cnt_scr[...], 1.0, None))[:, :NCLS]


def _zero_vec(ref, n):
    def body(i, _):
        ref[pl.ds(i * 16, 16)] = jnp.zeros((16,), jnp.float32)
        return 0
    lax.fori_loop(0, n // 16, body, 0)


def _rsqrt16(w):
    i = plsc.bitcast(w, jnp.int32)
    i = 0x5F3759DF - lax.shift_right_logical(i, 1)
    y = plsc.bitcast(i, jnp.float32)
    for _ in range(4):
        y = y * (1.5 - 0.5 * w * y * y)
    return y


def _wid_base():
    c = lax.axis_index("c")
    s = lax.axis_index("s")
    return c, s, (s * NC + c) * PAIRS_W


def _copy_idx(src_all, dst_small, j):
    for t in range(K // 16):
        dst_small[pl.ds(t * 16, 16)] = src_all[pl.ds(j * K + t * 16, 16)]


def _sc_counts(src_hbm, dst_hbm, cv_out, ce_out,
               sall, dall, s0, s1, d0, d1, ones_b, zb, cv_s, ce_s,
               sem0, sem1):
    c, s, base = _wid_base()
    pltpu.sync_copy(src_hbm.at[pl.ds(base, PAIRS_W)], sall)
    pltpu.sync_copy(dst_hbm.at[pl.ds(base, PAIRS_W)], dall)
    _zero_vec(zb, TV)

    def fill(i, _):
        ones_b[pl.ds(i * 16, 16)] = jnp.ones((16,), jnp.float32)
        return 0
    lax.fori_loop(0, K // 16, fill, 0)
    pltpu.sync_copy(zb, cv_s.at[pl.ds(s * TV, TV)])
    pltpu.sync_copy(zb.at[pl.ds(0, TE)], ce_s.at[pl.ds(s * TE, TE)])
    plsc.subcore_barrier()

    def issue(j, sb, db, sem):
        _copy_idx(sall, sb, j)
        _copy_idx(dall, db, j)
        pltpu.async_copy(ones_b, cv_s.at[sb], sem, add=True)
        pltpu.async_copy(ones_b, ce_s.at[db], sem, add=True)

    def drain(sb, db, sem):
        pltpu.make_async_copy(ones_b, cv_s.at[sb], sem).wait()
        pltpu.make_async_copy(ones_b, ce_s.at[db], sem).wait()

    issue(0, s0, d0, sem0)
    issue(1, s1, d1, sem1)

    def body(i, _):
        drain(s0, d0, sem0)
        issue(2 * i + 2, s0, d0, sem0)

        @pl.when(i < (NCHUNK - 3) // 2)
        def _():
            drain(s1, d1, sem1)
            issue(2 * i + 3, s1, d1, sem1)
        return 0
    lax.fori_loop(0, (NCHUNK - 1) // 2, body, 0)
    drain(s0, d0, sem0)
    drain(s1, d1, sem1)
    plsc.subcore_barrier()

    @pl.when(s == 0)
    def _():
        pltpu.sync_copy(cv_s, cv_out.at[pl.ds(c * NP, NP)])
        pltpu.sync_copy(ce_s, ce_out.at[pl.ds(c * EP, EP)])


def _zero_rows(rows):
    def zr(j, _):
        for k in range(8):
            rows[j, pl.ds(k * 16, 16)] = jnp.zeros((16,), jnp.float32)
        return 0
    lax.fori_loop(0, K, zr, 0)


def _sc_v2e(h_hbm, src_hbm, dst_hbm, xe_out,
            sall, dall, s0, s1, d0, d1, rows0, rows1, acc,
            semg0, semg1, sems0, sems1):
    c, s, base = _wid_base()
    pltpu.sync_copy(src_hbm.at[pl.ds(base, PAIRS_W)], sall)
    pltpu.sync_copy(dst_hbm.at[pl.ds(base, PAIRS_W)], dall)
    _zero_rows(rows0)
    for j in range(TE // K):
        pltpu.sync_copy(rows0, acc.at[pl.ds(s * TE + j * K, K)])
    plsc.subcore_barrier()

    def stage(j, sb, db, rows, semg):
        _copy_idx(sall, sb, j)
        _copy_idx(dall, db, j)
        pltpu.async_copy(h_hbm.at[sb], rows, semg)

    def scat(sb, db, rows, semg, sems):
        pltpu.make_async_copy(h_hbm.at[sb], rows, semg).wait()
        pltpu.async_copy(rows, acc.at[db], sems, add=True)

    def wscat(db, rows, sems):
        pltpu.make_async_copy(rows, acc.at[db], sems).wait()

    stage(0, s0, d0, rows0, semg0)
    stage(1, s1, d1, rows1, semg1)

    def body(i, _):
        scat(s0, d0, rows0, semg0, sems0)
        scat(s1, d1, rows1, semg1, sems1)
        wscat(d0, rows0, sems0)
        stage(2 * i + 2, s0, d0, rows0, semg0)

        @pl.when(i < (NCHUNK - 3) // 2)
        def _():
            wscat(d1, rows1, sems1)
            stage(2 * i + 3, s1, d1, rows1, semg1)
        return 0
    lax.fori_loop(0, (NCHUNK - 1) // 2, body, 0)
    scat(s0, d0, rows0, semg0, sems0)
    wscat(d0, rows0, sems0)
    wscat(d1, rows1, sems1)
    plsc.subcore_barrier()
    pltpu.sync_copy(acc.at[pl.ds(s * TE, TE)], xe_out.at[c, pl.ds(s * TE, TE)])


def _sc_stats(src_hbm, dst_hbm, sv_hbm, qv_hbm, se_hbm, qe_hbm,
              rall_out, r_out, mr_out,
              sall, dall, scb0, scb1, rbuf0, rbuf1, mbuf0, mbuf1, rtile,
              tsv, tqv, tse, tqe, racc, macc, sem0, sem1):
    c, s, base = _wid_base()
    pltpu.sync_copy(src_hbm.at[pl.ds(base, PAIRS_W)], sall)
    pltpu.sync_copy(dst_hbm.at[pl.ds(base, PAIRS_W)], dall)
    pltpu.sync_copy(sv_hbm, tsv)
    pltpu.sync_copy(qv_hbm, tqv)
    pltpu.sync_copy(se_hbm, tse)
    pltpu.sync_copy(qe_hbm, tqe)
    _zero_vec(rbuf0, K)
    for j in range(TV // K):
        pltpu.sync_copy(rbuf0, racc.at[pl.ds(s * TV + j * K, K)])
        pltpu.sync_copy(rbuf0, macc.at[pl.ds(s * TV + j * K, K)])
    plsc.subcore_barrier()

    def issue(i, scb, rbuf, mbuf, sem):
        for t in range(K // 16):
            sl = pl.ds(t * 16, 16)
            si = sall[pl.ds(i * K + t * 16, 16)]
            di = dall[pl.ds(i * K + t * 16, 16)]
            scb[sl] = si
            m = plsc.load_gather(tsv, [si]) + plsc.load_gather(tse, [di])
            w = (plsc.load_gather(tqv, [si]) + plsc.load_gather(tqe, [di])
                 - m * m + EPS)
            r = _rsqrt16(w)
            rbuf[sl] = r
            mbuf[sl] = m * r
            rtile[pl.ds(i * K + t * 16, 16)] = r
        pltpu.async_copy(rbuf, racc.at[scb], sem, add=True)
        pltpu.async_copy(mbuf, macc.at[scb], sem, add=True)

    def drain(scb, rbuf, mbuf, sem):
        pltpu.make_async_copy(rbuf, racc.at[scb], sem).wait()
        pltpu.make_async_copy(mbuf, macc.at[scb], sem).wait()

    issue(0, scb0, rbuf0, mbuf0, sem0)
    issue(1, scb1, rbuf1, mbuf1, sem1)

    def body(i, _):
        drain(scb0, rbuf0, mbuf0, sem0)
        issue(2 * i + 2, scb0, rbuf0, mbuf0, sem0)

        @pl.when(i < (NCHUNK - 3) // 2)
        def _():
            drain(scb1, rbuf1, mbuf1, sem1)
            issue(2 * i + 3, scb1, rbuf1, mbuf1, sem1)
        return 0
    lax.fori_loop(0, (NCHUNK - 1) // 2, body, 0)
    drain(scb0, rbuf0, mbuf0, sem0)
    drain(scb1, rbuf1, mbuf1, sem1)
    plsc.subcore_barrier()
    pltpu.sync_copy(rtile, rall_out.at[pl.ds(base, PAIRS_W)])

    @pl.when(s == 0)
    def _():
        pltpu.sync_copy(racc, r_out.at[pl.ds(c * NP, NP)])
        pltpu.sync_copy(macc, mr_out.at[pl.ds(c * NP, NP)])


def _sc_apply(b_hbm, src_hbm, dst_hbm, rall_hbm, sb_out,
              sall, dall, s0, s1, d0, d1, rc0, rc1, rows0, rows1,
              sbacc, semg0, semg1, sems0, sems1):
    c, s, base = _wid_base()
    pltpu.sync_copy(src_hbm.at[pl.ds(base, PAIRS_W)], sall)
    pltpu.sync_copy(dst_hbm.at[pl.ds(base, PAIRS_W)], dall)
    _zero_rows(rows0)
    for j in range(TV // K):
        pltpu.sync_copy(rows0, sbacc.at[pl.ds(s * TV + j * K, K)])
    plsc.subcore_barrier()

    def stage(j, sb, db, rc, rows, semg):
        _copy_idx(sall, sb, j)
        _copy_idx(dall, db, j)
        pltpu.async_copy(b_hbm.at[db], rows, semg)
        pltpu.async_copy(rall_hbm.at[pl.ds(base + j * K, K)], rc, semg)

    def scat(j, sb, db, rc, rows, semg, sems):
        pltpu.make_async_copy(b_hbm.at[db], rows, semg).wait()
        pltpu.make_async_copy(
            rall_hbm.at[pl.ds(base + j * K, K)], rc, semg).wait()

        def rowfn(jj, _):
            rj = plsc.load_gather(rc, [jnp.full((16,), jj, jnp.int32)])
            for k in range(8):
                sl = pl.ds(k * 16, 16)
                rows[jj, sl] = rows[jj, sl] * rj
            return 0
        lax.fori_loop(0, K, rowfn, 0)
        pltpu.async_copy(rows, sbacc.at[sb], sems, add=True)

    def wscat(sb, rows, sems):
        pltpu.make_async_copy(rows, sbacc.at[sb], sems).wait()

    stage(0, s0, d0, rc0, rows0, semg0)
    stage(1, s1, d1, rc1, rows1, semg1)

    def body(i, _):
        scat(2 * i, s0, d0, rc0, rows0, semg0, sems0)
        scat(2 * i + 1, s1, d1, rc1, rows1, semg1, sems1)
        wscat(s0, rows0, sems0)
        stage(2 * i + 2, s0, d0, rc0, rows0, semg0)

        @pl.when(i < (NCHUNK - 3) // 2)
        def _():
            wscat(s1, rows1, sems1)
            stage(2 * i + 3, s1, d1, rc1, rows1, semg1)
        return 0
    lax.fori_loop(0, (NCHUNK - 1) // 2, body, 0)
    scat(NCHUNK - 1, s0, d0, rc0, rows0, semg0, sems0)
    wscat(s0, rows0, sems0)
    wscat(s1, rows1, sems1)
    plsc.subcore_barrier()
    pltpu.sync_copy(sbacc.at[pl.ds(s * TV, TV)], sb_out.at[c, pl.ds(s * TV, TV)])


_IVEC = pltpu.VMEM((PAIRS_W,), jnp.int32)
_IK = pltpu.VMEM((K,), jnp.int32)
_FK = pltpu.VMEM((K,), jnp.float32)
_ROWS = pltpu.VMEM((K, C), jnp.float32)

_counts_call = functools.partial(
    pl.kernel, _sc_counts, mesh=_SC_MESH, compiler_params=_SC_PARAMS,
    out_type=[jax.ShapeDtypeStruct((NC * NP,), jnp.float32),
              jax.ShapeDtypeStruct((NC * EP,), jnp.float32)],
    scratch_types=[_IVEC, _IVEC, _IK, _IK, _IK, _IK, _FK,
                   pltpu.VMEM((TV,), jnp.float32),
                   pltpu.VMEM_SHARED((NP,), jnp.float32),
                   pltpu.VMEM_SHARED((EP,), jnp.float32),
                   pltpu.SemaphoreType.DMA, pltpu.SemaphoreType.DMA])

_v2e_call = functools.partial(
    pl.kernel, _sc_v2e, mesh=_SC_MESH, compiler_params=_SC_PARAMS,
    out_type=[jax.ShapeDtypeStruct((NC, EP, C), jnp.float32)],
    scratch_types=[_IVEC, _IVEC, _IK, _IK, _IK, _IK, _ROWS, _ROWS,
                   pltpu.VMEM_SHARED((EP, C), jnp.float32),
                   pltpu.SemaphoreType.DMA, pltpu.SemaphoreType.DMA,
                   pltpu.SemaphoreType.DMA, pltpu.SemaphoreType.DMA])

_stats_call = functools.partial(
    pl.kernel, _sc_stats, mesh=_SC_MESH, compiler_params=_SC_PARAMS,
    out_type=[jax.ShapeDtypeStruct((NNZ,), jnp.float32),
              jax.ShapeDtypeStruct((NC * NP,), jnp.float32),
              jax.ShapeDtypeStruct((NC * NP,), jnp.float32)],
    scratch_types=[_IVEC, _IVEC, _IK, _IK, _FK, _FK, _FK, _FK,
                   pltpu.VMEM((PAIRS_W,), jnp.float32),
                   pltpu.VMEM((NP,), jnp.float32), pltpu.VMEM((NP,), jnp.float32),
                   pltpu.VMEM((EP,), jnp.float32), pltpu.VMEM((EP,), jnp.float32),
                   pltpu.VMEM_SHARED((NP,), jnp.float32),
                   pltpu.VMEM_SHARED((NP,), jnp.float32),
                   pltpu.SemaphoreType.DMA, pltpu.SemaphoreType.DMA])

_apply_call = functools.partial(
    pl.kernel, _sc_apply, mesh=_SC_MESH, compiler_params=_SC_PARAMS,
    out_type=[jax.ShapeDtypeStruct((NC, NP, C), jnp.float32)],
    scratch_types=[_IVEC, _IVEC, _IK, _IK, _IK, _IK, _FK, _FK, _ROWS, _ROWS,
                   pltpu.VMEM_SHARED((NP, C), jnp.float32),
                   pltpu.SemaphoreType.DMA, pltpu.SemaphoreType.DMA,
                   pltpu.SemaphoreType.DMA, pltpu.SemaphoreType.DMA])


def _row_spec(blk):
    return pl.BlockSpec((blk, C), lambda i: (i, 0))


def _full(shape):
    return pl.BlockSpec(shape, lambda i: tuple(0 for _ in shape))


def _pad1(x, n):
    return jnp.pad(jnp.reshape(x, (-1,)), (0, n - x.shape[0]))


def kernel(X, src, dst, all_batch, lin_in_w, lin_in_b, w1_ln0_g, w1_ln0_b,
           w1_lin0_w, w1_lin0_b, w1_ln1_g, w1_ln1_b, w1_lin1_w, w1_lin1_b,
           w2_ln_g, w2_ln_b, w2_lin_w, w2_lin_b, w3_ln_g, w3_ln_b, w3_lin_w,
           w3_lin_b, clf_lin0_w, clf_lin0_b, clf_ln_g, clf_ln_b, clf_lin1_w,
           clf_lin1_b):
    f32 = jnp.float32
    row = lambda v: jnp.reshape(v, (1, -1))

    x0 = pl.pallas_call(
        _k_input,
        grid=(GRID,),
        in_specs=[_row_spec(ROWB), _full((C, C)), _full((1, C))],
        out_specs=_row_spec(ROWB),
        out_shape=jax.ShapeDtypeStruct((N, C), f32),
    )(X, lin_in_w, row(lin_in_b))

    gb = jnp.stack([w2_ln_g, w2_ln_b], axis=0)
    gwc = pl.pallas_call(
        _k_prep,
        grid=(1,),
        in_specs=[_full((2, 2 * C)), _full((2 * C, C)), _full((1, C))],
        out_specs=_full((2, C)),
        out_shape=jax.ShapeDtypeStruct((2, C), f32),
    )(gb, w2_lin_w, row(w2_lin_b))

    cv_p, ce_p = _counts_call()(src, dst)
    cv3 = jnp.reshape(cv_p, (NC, NP))[:, :N, None]
    ce3 = jnp.reshape(ce_p, (NC, EP))[:, :EH, None]

    w2top = w2_lin_w[:C]
    w2bot = w2_lin_w[C:]
    g2lo = row(w2_ln_g[:C])
    g2hi = row(w2_ln_g[C:])

    x = x0
    for _ in range(NLAYER):
        h2, a_mat, sv, qv = pl.pallas_call(
            _k_layer_a,
            grid=(GRID,),
            in_specs=[_row_spec(ROWB)] + [_full((1, C))] * 2
            + [_full((C, C)), _full((1, C))] + [_full((1, C))] * 2
            + [_full((C, C)), _full((1, C))] + [_full((1, C)), _full((C, C))],
            out_specs=[_row_spec(ROWB), _row_spec(ROWB),
                       pl.BlockSpec((ROWB, 1), lambda i: (i, 0)),
                       pl.BlockSpec((ROWB, 1), lambda i: (i, 0))],
            out_shape=[jax.ShapeDtypeStruct((N, C), f32),
                       jax.ShapeDtypeStruct((N, C), f32),
                       jax.ShapeDtypeStruct((N, 1), f32),
                       jax.ShapeDtypeStruct((N, 1), f32)],
        )(x, row(w1_ln0_g), row(w1_ln0_b), w1_lin0_w, row(w1_lin0_b),
          row(w1_ln1_g), row(w1_ln1_b), w1_lin1_w, row(w1_lin1_b),
          g2lo, w2top)

        (xe_p,) = _v2e_call()(h2, src, dst)

        b_mat, se, qe = pl.pallas_call(
            _k_edge,
            grid=(EGRID,),
            in_specs=[pl.BlockSpec((NC, EROWB, C), lambda i: (0, i, 0)),
                      pl.BlockSpec((NC, EROWB, 1), lambda i: (0, i, 0)),
                      _full((1, C)), _full((C, C))],
            out_specs=[_row_spec(EROWB),
                       pl.BlockSpec((EROWB, 1), lambda i: (i, 0)),
                       pl.BlockSpec((EROWB, 1), lambda i: (i, 0))],
            out_shape=[jax.ShapeDtypeStruct((EH, C), f32),
                       jax.ShapeDtypeStruct((EH, 1), f32),
                       jax.ShapeDtypeStruct((EH, 1), f32)],
        )(xe_p[:, :EH], ce3, g2hi, w2bot)

        rall, r_p, mr_p = _stats_call()(
            src, dst, _pad1(sv, NP), _pad1(qv, NP),
            _pad1(se, EP), _pad1(qe, EP))
        (sb_p,) = _apply_call()(b_mat, src, dst, rall)

        x = pl.pallas_call(
            _k_vertex,
            grid=(GRID,),
            in_specs=[pl.BlockSpec((NC, ROWB, C), lambda i: (0, i, 0)),
                      pl.BlockSpec((NC, ROWB, 1), lambda i: (0, i, 0)),
                      pl.BlockSpec((NC, ROWB, 1), lambda i: (0, i, 0)),
                      pl.BlockSpec((NC, ROWB, 1), lambda i: (0, i, 0)),
                      _row_spec(ROWB), _row_spec(ROWB), _full((2, C)),
                      _full((1, C)), _full((1, C)), _full((C, C)), _full((1, C))],
            out_specs=_row_spec(ROWB),
            out_shape=jax.ShapeDtypeStruct((N, C), f32),
        )(sb_p[:, :N], jnp.reshape(r_p, (NC, NP))[:, :N, None],
          jnp.reshape(mr_p, (NC, NP))[:, :N, None], cv3, a_mat, x0, gwc,
          row(w3_ln_g), row(w3_ln_b), w3_lin_w, row(w3_lin_b))

    wc1p = jnp.pad(clf_lin1_w, ((0, 0), (0, C - NCLS)))
    bc1p = row(jnp.pad(clf_lin1_b, (0, C - NCLS)))
    ab3 = jnp.reshape(all_batch.astype(jnp.int32), (GRID, 1, ROWB))
    readout = pl.pallas_call(
        _k_clf,
        grid=(GRID,),
        in_specs=[_row_spec(ROWB), _full((C, C)), _full((1, C)),
                  _full((1, C)), _full((1, C)), _full((C, C)), _full((1, C)),
                  pl.BlockSpec((1, 1, ROWB), lambda i: (i, 0, 0))],
        out_specs=_full((NGRAPH, NCLS)),
        out_shape=jax.ShapeDtypeStruct((NGRAPH, NCLS), f32),
        scratch_shapes=[pltpu.VMEM((NGRAPH, C), f32),
                        pltpu.VMEM((NGRAPH, C), f32)],
    )(x, clf_lin0_w, row(clf_lin0_b), row(clf_ln_g), row(clf_ln_b),
      wc1p, bc1p, ab3)
    return readout


# speedup vs baseline: 1.1134x; 1.1134x over previous
import functools

import jax
import jax.numpy as jnp
from jax import lax
from jax.experimental import pallas as pl
from jax.experimental.pallas import tpu as pltpu
from jax.experimental.pallas import tpu_sc as plsc

N, EH, NNZ, C, NCLS, NGRAPH, NLAYER, ALPHA = 10000, 5000, 320000, 128, 10, 16, 2, 0.5
EPS = 1e-5
NP = 10240
EP = 5120
NC, NS = 2, 16
NW = NC * NS
PAIRS_W = NNZ // NW
K = 80
NCHUNK = PAIRS_W // K
TV = NP // NS
TE = EP // NS
ROWB = 2000
GRID = N // ROWB
EROWB = 1000
EGRID = EH // EROWB

_SC_MESH = plsc.VectorSubcoreMesh(core_axis_name="c", subcore_axis_name="s")
_SC_PARAMS = pltpu.CompilerParams(needs_layout_passes=False)


def _ln(x, g, b):
    m = jnp.mean(x, axis=-1, keepdims=True)
    v = jnp.mean((x - m) ** 2, axis=-1, keepdims=True)
    return (x - m) * lax.rsqrt(v + EPS) * g + b


def _dot(a, b):
    return jnp.dot(a, b, preferred_element_type=jnp.float32)


def _k_input(x_ref, w_ref, b_ref, o_ref):
    o_ref[...] = jnp.maximum(_dot(x_ref[...], w_ref[...]) + b_ref[...], 0.0)


def _k_prep(gb_ref, w2_ref, b2_ref, o_ref):
    o = _dot(gb_ref[...], w2_ref[...])
    o_ref[...] = o + jnp.concatenate(
        [jnp.zeros_like(b2_ref[...]), b2_ref[...]], axis=0)


def _k_layer_a(x_ref, g0, b0, w10, b10, g1, b1, w11, b11, g2lo, w2top,
               h2_o, a_o, sv_o, qv_o):
    x = x_ref[...]
    h = _ln(x, g0[...], b0[...])
    h = jnp.maximum(_dot(h, w10[...]) + b10[...], 0.0)
    h = _ln(h, g1[...], b1[...])
    h2_o[...] = _dot(h, w11[...]) + b11[...]
    a_o[...] = _dot(x * g2lo[...], w2top[...])
    sv_o[...] = jnp.sum(x, axis=-1, keepdims=True) * (1.0 / (2 * C))
    qv_o[...] = jnp.sum(x * x, axis=-1, keepdims=True) * (1.0 / (2 * C))


def _k_edge(xs_ref, ce_ref, g2hi, w2bot, b_o, se_o, qe_o):
    xs = xs_ref[...]
    cnt = ce_ref[0] + ce_ref[1]
    xe = (xs[0] + xs[1]) / jnp.clip(cnt, 1.0, None)
    b_o[...] = _dot(xe * g2hi[...], w2bot[...])
    se_o[...] = jnp.sum(xe, axis=-1, keepdims=True) * (1.0 / (2 * C))
    qe_o[...] = jnp.sum(xe * xe, axis=-1, keepdims=True) * (1.0 / (2 * C))


def _k_vertex(sb_ref, r_ref, mr_ref, cv_ref, a_ref, x0_ref, gwc_ref,
              g3, b3, w3, b3l, x_o):
    cnt = cv_ref[0] + cv_ref[1]
    gw = gwc_ref[0:1]
    bwc = gwc_ref[1:2]
    seg = (a_ref[...] * (r_ref[0] + r_ref[1])
           + (sb_ref[0] + sb_ref[1])
           - (mr_ref[0] + mr_ref[1]) * gw
           + cnt * bwc)
    xv = seg / jnp.clip(cnt, 1.0, None)
    xn = (1.0 - ALPHA) * xv + ALPHA * x0_ref[...]
    x_o[...] = jnp.maximum(_dot(_ln(xn, g3[...], b3[...]), w3[...]) + b3l[...], 0.0)


def _k_clf(x_ref, wc0, bc0, gc, bc, wc1, bc1, ab_ref, out_ref, sum_scr, cnt_scr):
    i = pl.program_id(0)

    @pl.when(i == 0)
    def _():
        sum_scr[...] = jnp.zeros_like(sum_scr)
        cnt_scr[...] = jnp.zeros_like(cnt_scr)

    h = jnp.maximum(_dot(x_ref[...], wc0[...]) + bc0[...], 0.0)
    h = _ln(h, gc[...], bc[...])
    o = _dot(h, wc1[...]) + bc1[...]
    ab = ab_ref[0]
    gids = lax.broadcasted_iota(jnp.int32, (NGRAPH, ROWB), 0)
    oh = jnp.where(ab == gids, 1.0, 0.0)
    sum_scr[...] += _dot(oh, o)
    cnt_scr[...] += jnp.broadcast_to(
        jnp.sum(oh, axis=-1, keepdims=True), (NGRAPH, C))

    @pl.when(i == GRID - 1)
    def _():
        out_ref[...] = (sum_scr[...] / jnp.clip(cnt_scr[...], 1.0, None))[:, :NCLS]


def _zero_vec(ref, n):
    def body(i, _):
        ref[pl.ds(i * 16, 16)] = jnp.zeros((16,), jnp.float32)
        return 0
    lax.fori_loop(0, n // 16, body, 0)


def _rsqrt16(w):
    i = plsc.bitcast(w, jnp.int32)
    i = 0x5F3759DF - lax.shift_right_logical(i, 1)
    y = plsc.bitcast(i, jnp.float32)
    for _ in range(4):
        y = y * (1.5 - 0.5 * w * y * y)
    return y


def _wid_base():
    c = lax.axis_index("c")
    s = lax.axis_index("s")
    return c, s, (s * NC + c) * PAIRS_W


def _copy_idx(src_all, dst_small, j):
    for t in range(K // 16):
        dst_small[pl.ds(t * 16, 16)] = src_all[pl.ds(j * K + t * 16, 16)]


def _sc_counts(src_hbm, dst_hbm, cv_out, ce_out,
               sall, dall, s0, s1, d0, d1, ones_b, zb, cv_s, ce_s,
               sem0, sem1):
    c, s, base = _wid_base()
    pltpu.sync_copy(src_hbm.at[pl.ds(base, PAIRS_W)], sall)
    pltpu.sync_copy(dst_hbm.at[pl.ds(base, PAIRS_W)], dall)
    _zero_vec(zb, TV)

    def fill(i, _):
        ones_b[pl.ds(i * 16, 16)] = jnp.ones((16,), jnp.float32)
        return 0
    lax.fori_loop(0, K // 16, fill, 0)
    pltpu.sync_copy(zb, cv_s.at[pl.ds(s * TV, TV)])
    pltpu.sync_copy(zb.at[pl.ds(0, TE)], ce_s.at[pl.ds(s * TE, TE)])
    plsc.subcore_barrier()

    def issue(j, sb, db, sem):
        _copy_idx(sall, sb, j)
        _copy_idx(dall, db, j)
        pltpu.async_copy(ones_b, cv_s.at[sb], sem, add=True)
        pltpu.async_copy(ones_b, ce_s.at[db], sem, add=True)

    def drain(sb, db, sem):
        pltpu.make_async_copy(ones_b, cv_s.at[sb], sem).wait()
        pltpu.make_async_copy(ones_b, ce_s.at[db], sem).wait()

    issue(0, s0, d0, sem0)
    issue(1, s1, d1, sem1)

    def body(i, _):
        drain(s0, d0, sem0)
        issue(2 * i + 2, s0, d0, sem0)

        @pl.when(i < (NCHUNK - 3) // 2)
        def _():
            drain(s1, d1, sem1)
            issue(2 * i + 3, s1, d1, sem1)
        return 0
    lax.fori_loop(0, (NCHUNK - 1) // 2, body, 0)
    drain(s0, d0, sem0)
    drain(s1, d1, sem1)
    plsc.subcore_barrier()

    @pl.when(s == 0)
    def _():
        pltpu.sync_copy(cv_s, cv_out.at[pl.ds(c * NP, NP)])
        pltpu.sync_copy(ce_s, ce_out.at[pl.ds(c * EP, EP)])


def _zero_rows(rows):
    def zr(j, _):
        for k in range(8):
            rows[j, pl.ds(k * 16, 16)] = jnp.zeros((16,), jnp.float32)
        return 0
    lax.fori_loop(0, K, zr, 0)


def _sc_v2e(h_hbm, src_hbm, dst_hbm, xe_out,
            sall, dall, s0, s1, d0, d1, rows0, rows1, acc,
            semg0, semg1, sems0, sems1):
    c, s, base = _wid_base()
    pltpu.sync_copy(src_hbm.at[pl.ds(base, PAIRS_W)], sall)
    pltpu.sync_copy(dst_hbm.at[pl.ds(base, PAIRS_W)], dall)
    _zero_rows(rows0)
    for j in range(TE // K):
        pltpu.sync_copy(rows0, acc.at[pl.ds(s * TE + j * K, K)])
    plsc.subcore_barrier()

    def stage(j, sb, db, rows, semg):
        _copy_idx(sall, sb, j)
        _copy_idx(dall, db, j)
        pltpu.async_copy(h_hbm.at[sb], rows, semg)

    def finish(sb, db, rows, semg):
        pltpu.make_async_copy(h_hbm.at[sb], rows, semg).wait()
        pltpu.sync_copy(rows, acc.at[db], add=True)

    stage(0, s0, d0, rows0, semg0)

    def body(i, _):
        stage(2 * i + 1, s1, d1, rows1, semg1)
        finish(s0, d0, rows0, semg0)
        stage(2 * i + 2, s0, d0, rows0, semg0)
        finish(s1, d1, rows1, semg1)
        return 0
    lax.fori_loop(0, (NCHUNK - 1) // 2, body, 0)
    finish(s0, d0, rows0, semg0)
    plsc.subcore_barrier()
    pltpu.sync_copy(acc.at[pl.ds(s * TE, TE)], xe_out.at[c, pl.ds(s * TE, TE)])


def _sc_stats(src_hbm, dst_hbm, sv_hbm, qv_hbm, se_hbm, qe_hbm,
              rall_out, r_out, mr_out,
              sall, dall, scb0, scb1, rbuf0, rbuf1, mbuf0, mbuf1, rtile,
              tsv, tqv, tse, tqe, racc, macc, sem0, sem1):
    c, s, base = _wid_base()
    pltpu.sync_copy(src_hbm.at[pl.ds(base, PAIRS_W)], sall)
    pltpu.sync_copy(dst_hbm.at[pl.ds(base, PAIRS_W)], dall)
    pltpu.sync_copy(sv_hbm, tsv)
    pltpu.sync_copy(qv_hbm, tqv)
    pltpu.sync_copy(se_hbm, tse)
    pltpu.sync_copy(qe_hbm, tqe)
    _zero_vec(rbuf0, K)
    for j in range(TV // K):
        pltpu.sync_copy(rbuf0, racc.at[pl.ds(s * TV + j * K, K)])
        pltpu.sync_copy(rbuf0, macc.at[pl.ds(s * TV + j * K, K)])
    plsc.subcore_barrier()

    def issue(i, scb, rbuf, mbuf, sem):
        for t in range(K // 16):
            sl = pl.ds(t * 16, 16)
            si = sall[pl.ds(i * K + t * 16, 16)]
            di = dall[pl.ds(i * K + t * 16, 16)]
            scb[sl] = si
            m = plsc.load_gather(tsv, [si]) + plsc.load_gather(tse, [di])
            w = (plsc.load_gather(tqv, [si]) + plsc.load_gather(tqe, [di])
                 - m * m + EPS)
            r = _rsqrt16(w)
            rbuf[sl] = r
            mbuf[sl] = m * r
            rtile[pl.ds(i * K + t * 16, 16)] = r
        pltpu.async_copy(rbuf, racc.at[scb], sem, add=True)
        pltpu.async_copy(mbuf, macc.at[scb], sem, add=True)

    def drain(scb, rbuf, mbuf, sem):
        pltpu.make_async_copy(rbuf, racc.at[scb], sem).wait()
        pltpu.make_async_copy(mbuf, macc.at[scb], sem).wait()

    issue(0, scb0, rbuf0, mbuf0, sem0)
    issue(1, scb1, rbuf1, mbuf1, sem1)

    def body(i, _):
        drain(scb0, rbuf0, mbuf0, sem0)
        issue(2 * i + 2, scb0, rbuf0, mbuf0, sem0)

        @pl.when(i < (NCHUNK - 3) // 2)
        def _():
            drain(scb1, rbuf1, mbuf1, sem1)
            issue(2 * i + 3, scb1, rbuf1, mbuf1, sem1)
        return 0
    lax.fori_loop(0, (NCHUNK - 1) // 2, body, 0)
    drain(scb0, rbuf0, mbuf0, sem0)
    drain(scb1, rbuf1, mbuf1, sem1)
    plsc.subcore_barrier()
    pltpu.sync_copy(rtile, rall_out.at[pl.ds(base, PAIRS_W)])

    @pl.when(s == 0)
    def _():
        pltpu.sync_copy(racc, r_out.at[pl.ds(c * NP, NP)])
        pltpu.sync_copy(macc, mr_out.at[pl.ds(c * NP, NP)])


def _sc_apply(b_hbm, src_hbm, dst_hbm, rall_hbm, sb_out,
              sall, dall, s0, s1, d0, d1, rc0, rc1, rows0, rows1,
              sbacc, semg0, semg1, sems0, sems1):
    c, s, base = _wid_base()
    pltpu.sync_copy(src_hbm.at[pl.ds(base, PAIRS_W)], sall)
    pltpu.sync_copy(dst_hbm.at[pl.ds(base, PAIRS_W)], dall)
    _zero_rows(rows0)
    for j in range(TV // K):
        pltpu.sync_copy(rows0, sbacc.at[pl.ds(s * TV + j * K, K)])
    plsc.subcore_barrier()

    def stage(j, sb, db, rc, rows, semg):
        _copy_idx(sall, sb, j)
        _copy_idx(dall, db, j)
        pltpu.async_copy(b_hbm.at[db], rows, semg)
        pltpu.async_copy(rall_hbm.at[pl.ds(base + j * K, K)], rc, semg)

    def finish(j, sb, db, rc, rows, semg):
        pltpu.make_async_copy(b_hbm.at[db], rows, semg).wait()
        pltpu.make_async_copy(
            rall_hbm.at[pl.ds(base + j * K, K)], rc, semg).wait()

        def rowfn(jj, _):
            rj = plsc.load_gather(rc, [jnp.full((16,), jj, jnp.int32)])
            for k in range(8):
                sl = pl.ds(k * 16, 16)
                rows[jj, sl] = rows[jj, sl] * rj
            return 0
        lax.fori_loop(0, K, rowfn, 0)
        pltpu.sync_copy(rows, sbacc.at[sb], add=True)

    stage(0, s0, d0, rc0, rows0, semg0)

    def body(i, _):
        stage(2 * i + 1, s1, d1, rc1, rows1, semg1)
        finish(2 * i, s0, d0, rc0, rows0, semg0)
        stage(2 * i + 2, s0, d0, rc0, rows0, semg0)
        finish(2 * i + 1, s1, d1, rc1, rows1, semg1)
        return 0
    lax.fori_loop(0, (NCHUNK - 1) // 2, body, 0)
    finish(NCHUNK - 1, s0, d0, rc0, rows0, semg0)
    plsc.subcore_barrier()
    pltpu.sync_copy(sbacc.at[pl.ds(s * TV, TV)], sb_out.at[c, pl.ds(s * TV, TV)])


_IVEC = pltpu.VMEM((PAIRS_W,), jnp.int32)
_IK = pltpu.VMEM((K,), jnp.int32)
_FK = pltpu.VMEM((K,), jnp.float32)
_ROWS = pltpu.VMEM((K, C), jnp.float32)

_counts_call = functools.partial(
    pl.kernel, _sc_counts, mesh=_SC_MESH, compiler_params=_SC_PARAMS,
    out_type=[jax.ShapeDtypeStruct((NC * NP,), jnp.float32),
              jax.ShapeDtypeStruct((NC * EP,), jnp.float32)],
    scratch_types=[_IVEC, _IVEC, _IK, _IK, _IK, _IK, _FK,
                   pltpu.VMEM((TV,), jnp.float32),
                   pltpu.VMEM_SHARED((NP,), jnp.float32),
                   pltpu.VMEM_SHARED((EP,), jnp.float32),
                   pltpu.SemaphoreType.DMA, pltpu.SemaphoreType.DMA])

_v2e_call = functools.partial(
    pl.kernel, _sc_v2e, mesh=_SC_MESH, compiler_params=_SC_PARAMS,
    out_type=[jax.ShapeDtypeStruct((NC, EP, C), jnp.float32)],
    scratch_types=[_IVEC, _IVEC, _IK, _IK, _IK, _IK, _ROWS, _ROWS,
                   pltpu.VMEM_SHARED((EP, C), jnp.float32),
                   pltpu.SemaphoreType.DMA, pltpu.SemaphoreType.DMA,
                   pltpu.SemaphoreType.DMA, pltpu.SemaphoreType.DMA])

_stats_call = functools.partial(
    pl.kernel, _sc_stats, mesh=_SC_MESH, compiler_params=_SC_PARAMS,
    out_type=[jax.ShapeDtypeStruct((NNZ,), jnp.float32),
              jax.ShapeDtypeStruct((NC * NP,), jnp.float32),
              jax.ShapeDtypeStruct((NC * NP,), jnp.float32)],
    scratch_types=[_IVEC, _IVEC, _IK, _IK, _FK, _FK, _FK, _FK,
                   pltpu.VMEM((PAIRS_W,), jnp.float32),
                   pltpu.VMEM((NP,), jnp.float32), pltpu.VMEM((NP,), jnp.float32),
                   pltpu.VMEM((EP,), jnp.float32), pltpu.VMEM((EP,), jnp.float32),
                   pltpu.VMEM_SHARED((NP,), jnp.float32),
                   pltpu.VMEM_SHARED((NP,), jnp.float32),
                   pltpu.SemaphoreType.DMA, pltpu.SemaphoreType.DMA])

_apply_call = functools.partial(
    pl.kernel, _sc_apply, mesh=_SC_MESH, compiler_params=_SC_PARAMS,
    out_type=[jax.ShapeDtypeStruct((NC, NP, C), jnp.float32)],
    scratch_types=[_IVEC, _IVEC, _IK, _IK, _IK, _IK, _FK, _FK, _ROWS, _ROWS,
                   pltpu.VMEM_SHARED((NP, C), jnp.float32),
                   pltpu.SemaphoreType.DMA, pltpu.SemaphoreType.DMA,
                   pltpu.SemaphoreType.DMA, pltpu.SemaphoreType.DMA])


def _row_spec(blk):
    return pl.BlockSpec((blk, C), lambda i: (i, 0))


def _full(shape):
    return pl.BlockSpec(shape, lambda i: tuple(0 for _ in shape))


def _pad1(x, n):
    return jnp.pad(jnp.reshape(x, (-1,)), (0, n - x.shape[0]))


def kernel(X, src, dst, all_batch, lin_in_w, lin_in_b, w1_ln0_g, w1_ln0_b,
           w1_lin0_w, w1_lin0_b, w1_ln1_g, w1_ln1_b, w1_lin1_w, w1_lin1_b,
           w2_ln_g, w2_ln_b, w2_lin_w, w2_lin_b, w3_ln_g, w3_ln_b, w3_lin_w,
           w3_lin_b, clf_lin0_w, clf_lin0_b, clf_ln_g, clf_ln_b, clf_lin1_w,
           clf_lin1_b):
    f32 = jnp.float32
    row = lambda v: jnp.reshape(v, (1, -1))

    x0 = pl.pallas_call(
        _k_input,
        grid=(GRID,),
        in_specs=[_row_spec(ROWB), _full((C, C)), _full((1, C))],
        out_specs=_row_spec(ROWB),
        out_shape=jax.ShapeDtypeStruct((N, C), f32),
    )(X, lin_in_w, row(lin_in_b))

    gb = jnp.stack([w2_ln_g, w2_ln_b], axis=0)
    gwc = pl.pallas_call(
        _k_prep,
        grid=(1,),
        in_specs=[_full((2, 2 * C)), _full((2 * C, C)), _full((1, C))],
        out_specs=_full((2, C)),
        out_shape=jax.ShapeDtypeStruct((2, C), f32),
    )(gb, w2_lin_w, row(w2_lin_b))

    cv_p, ce_p = _counts_call()(src, dst)
    cv3 = jnp.reshape(cv_p, (NC, NP))[:, :N, None]
    ce3 = jnp.reshape(ce_p, (NC, EP))[:, :EH, None]

    w2top = w2_lin_w[:C]
    w2bot = w2_lin_w[C:]
    g2lo = row(w2_ln_g[:C])
    g2hi = row(w2_ln_g[C:])

    x = x0
    for _ in range(NLAYER):
        h2, a_mat, sv, qv = pl.pallas_call(
            _k_layer_a,
            grid=(GRID,),
            in_specs=[_row_spec(ROWB)] + [_full((1, C))] * 2
            + [_full((C, C)), _full((1, C))] + [_full((1, C))] * 2
            + [_full((C, C)), _full((1, C))] + [_full((1, C)), _full((C, C))],
            out_specs=[_row_spec(ROWB), _row_spec(ROWB),
                       pl.BlockSpec((ROWB, 1), lambda i: (i, 0)),
                       pl.BlockSpec((ROWB, 1), lambda i: (i, 0))],
            out_shape=[jax.ShapeDtypeStruct((N, C), f32),
                       jax.ShapeDtypeStruct((N, C), f32),
                       jax.ShapeDtypeStruct((N, 1), f32),
                       jax.ShapeDtypeStruct((N, 1), f32)],
        )(x, row(w1_ln0_g), row(w1_ln0_b), w1_lin0_w, row(w1_lin0_b),
          row(w1_ln1_g), row(w1_ln1_b), w1_lin1_w, row(w1_lin1_b),
          g2lo, w2top)

        (xe_p,) = _v2e_call()(h2, src, dst)

        b_mat, se, qe = pl.pallas_call(
            _k_edge,
            grid=(EGRID,),
            in_specs=[pl.BlockSpec((NC, EROWB, C), lambda i: (0, i, 0)),
                      pl.BlockSpec((NC, EROWB, 1), lambda i: (0, i, 0)),
                      _full((1, C)), _full((C, C))],
            out_specs=[_row_spec(EROWB),
                       pl.BlockSpec((EROWB, 1), lambda i: (i, 0)),
                       pl.BlockSpec((EROWB, 1), lambda i: (i, 0))],
            out_shape=[jax.ShapeDtypeStruct((EH, C), f32),
                       jax.ShapeDtypeStruct((EH, 1), f32),
                       jax.ShapeDtypeStruct((EH, 1), f32)],
        )(xe_p[:, :EH], ce3, g2hi, w2bot)

        rall, r_p, mr_p = _stats_call()(
            src, dst, _pad1(sv, NP), _pad1(qv, NP),
            _pad1(se, EP), _pad1(qe, EP))
        (sb_p,) = _apply_call()(b_mat, src, dst, rall)

        x = pl.pallas_call(
            _k_vertex,
            grid=(GRID,),
            in_specs=[pl.BlockSpec((NC, ROWB, C), lambda i: (0, i, 0)),
                      pl.BlockSpec((NC, ROWB, 1), lambda i: (0, i, 0)),
                      pl.BlockSpec((NC, ROWB, 1), lambda i: (0, i, 0)),
                      pl.BlockSpec((NC, ROWB, 1), lambda i: (0, i, 0)),
                      _row_spec(ROWB), _row_spec(ROWB), _full((2, C)),
                      _full((1, C)), _full((1, C)), _full((C, C)), _full((1, C))],
            out_specs=_row_spec(ROWB),
            out_shape=jax.ShapeDtypeStruct((N, C), f32),
        )(sb_p[:, :N], jnp.reshape(r_p, (NC, NP))[:, :N, None],
          jnp.reshape(mr_p, (NC, NP))[:, :N, None], cv3, a_mat, x0, gwc,
          row(w3_ln_g), row(w3_ln_b), w3_lin_w, row(w3_lin_b))

    wc1p = jnp.pad(clf_lin1_w, ((0, 0), (0, C - NCLS)))
    bc1p = row(jnp.pad(clf_lin1_b, (0, C - NCLS)))
    ab3 = jnp.reshape(all_batch.astype(jnp.int32), (GRID, 1, ROWB))
    readout = pl.pallas_call(
        _k_clf,
        grid=(GRID,),
        in_specs=[_row_spec(ROWB), _full((C, C)), _full((1, C)),
                  _full((1, C)), _full((1, C)), _full((C, C)), _full((1, C)),
                  pl.BlockSpec((1, 1, ROWB), lambda i: (i, 0, 0))],
        out_specs=_full((NGRAPH, NCLS)),
        out_shape=jax.ShapeDtypeStruct((NGRAPH, NCLS), f32),
        scratch_shapes=[pltpu.VMEM((NGRAPH, C), f32),
                        pltpu.VMEM((NGRAPH, C), f32)],
    )(x, clf_lin0_w, row(clf_lin0_b), row(clf_ln_g), row(clf_ln_b),
      wc1p, bc1p, ab3)
    return readout


# speedup vs baseline: 1.1399x; 1.0239x over previous
import functools

import jax
import jax.numpy as jnp
from jax import lax
from jax.experimental import pallas as pl
from jax.experimental.pallas import tpu as pltpu
from jax.experimental.pallas import tpu_sc as plsc

N, EH, NNZ, C, NCLS, NGRAPH, NLAYER, ALPHA = 10000, 5000, 320000, 128, 10, 16, 2, 0.5
EPS = 1e-5
NP = 10240
EP = 5120
NC, NS = 2, 16
NW = NC * NS
PAIRS_W = NNZ // NW
K = 80
NCHUNK = PAIRS_W // K
TV = NP // NS
TE = EP // NS
ROWB = 2000
GRID = N // ROWB
EROWB = 1000
EGRID = EH // EROWB

_SC_MESH = plsc.VectorSubcoreMesh(core_axis_name="c", subcore_axis_name="s")
_SC_PARAMS = pltpu.CompilerParams(needs_layout_passes=False)


def _ln(x, g, b):
    m = jnp.mean(x, axis=-1, keepdims=True)
    v = jnp.mean((x - m) ** 2, axis=-1, keepdims=True)
    return (x - m) * lax.rsqrt(v + EPS) * g + b


def _dot(a, b):
    return jnp.dot(a, b, preferred_element_type=jnp.float32)


def _k_input(x_ref, w_ref, b_ref, o_ref):
    o_ref[...] = jnp.maximum(_dot(x_ref[...], w_ref[...]) + b_ref[...], 0.0)


def _k_prep(gb_ref, w2_ref, b2_ref, o_ref):
    o = _dot(gb_ref[...], w2_ref[...])
    o_ref[...] = o + jnp.concatenate(
        [jnp.zeros_like(b2_ref[...]), b2_ref[...]], axis=0)


def _k_layer_a(x_ref, g0, b0, w10, b10, g1, b1, w11, b11, g2lo, w2top,
               h2_o, a_o, sv_o, qv_o):
    x = x_ref[...]
    h = _ln(x, g0[...], b0[...])
    h = jnp.maximum(_dot(h, w10[...]) + b10[...], 0.0)
    h = _ln(h, g1[...], b1[...])
    h2_o[...] = _dot(h, w11[...]) + b11[...]
    a_o[...] = _dot(x * g2lo[...], w2top[...])
    sv_o[...] = jnp.sum(x, axis=-1, keepdims=True) * (1.0 / (2 * C))
    qv_o[...] = jnp.sum(x * x, axis=-1, keepdims=True) * (1.0 / (2 * C))


def _k_edge(xs_ref, ce_ref, g2hi, w2bot, b_o, se_o, qe_o):
    xs = xs_ref[...]
    cnt = ce_ref[0] + ce_ref[1]
    xe = (xs[0] + xs[1]) / jnp.clip(cnt, 1.0, None)
    b_o[...] = _dot(xe * g2hi[...], w2bot[...])
    se_o[...] = jnp.sum(xe, axis=-1, keepdims=True) * (1.0 / (2 * C))
    qe_o[...] = jnp.sum(xe * xe, axis=-1, keepdims=True) * (1.0 / (2 * C))


def _k_vertex(sb_ref, r_ref, mr_ref, cv_ref, a_ref, x0_ref, gwc_ref,
              g3, b3, w3, b3l, x_o):
    cnt = cv_ref[0] + cv_ref[1]
    gw = gwc_ref[0:1]
    bwc = gwc_ref[1:2]
    seg = (a_ref[...] * (r_ref[0] + r_ref[1])
           + (sb_ref[0] + sb_ref[1])
           - (mr_ref[0] + mr_ref[1]) * gw
           + cnt * bwc)
    xv = seg / jnp.clip(cnt, 1.0, None)
    xn = (1.0 - ALPHA) * xv + ALPHA * x0_ref[...]
    x_o[...] = jnp.maximum(_dot(_ln(xn, g3[...], b3[...]), w3[...]) + b3l[...], 0.0)


def _k_clf(x_ref, wc0, bc0, gc, bc, wc1, bc1, ab_ref, out_ref, sum_scr, cnt_scr):
    i = pl.program_id(0)

    @pl.when(i == 0)
    def _():
        sum_scr[...] = jnp.zeros_like(sum_scr)
        cnt_scr[...] = jnp.zeros_like(cnt_scr)

    h = jnp.maximum(_dot(x_ref[...], wc0[...]) + bc0[...], 0.0)
    h = _ln(h, gc[...], bc[...])
    o = _dot(h, wc1[...]) + bc1[...]
    ab = ab_ref[0]
    gids = lax.broadcasted_iota(jnp.int32, (NGRAPH, ROWB), 0)
    oh = jnp.where(ab == gids, 1.0, 0.0)
    sum_scr[...] += _dot(oh, o)
    cnt_scr[...] += jnp.broadcast_to(
        jnp.sum(oh, axis=-1, keepdims=True), (NGRAPH, C))

    @pl.when(i == GRID - 1)
    def _():
        out_ref[...] = (sum_scr[...] / jnp.clip(cnt_scr[...], 1.0, None))[:, :NCLS]


def _zero_vec(ref, n):
    def body(i, _):
        ref[pl.ds(i * 16, 16)] = jnp.zeros((16,), jnp.float32)
        return 0
    lax.fori_loop(0, n // 16, body, 0)


def _rsqrt16(w):
    i = plsc.bitcast(w, jnp.int32)
    i = 0x5F3759DF - lax.shift_right_logical(i, 1)
    y = plsc.bitcast(i, jnp.float32)
    for _ in range(4):
        y = y * (1.5 - 0.5 * w * y * y)
    return y


def _wid_base():
    c = lax.axis_index("c")
    s = lax.axis_index("s")
    return c, s, (s * NC + c) * PAIRS_W


def _copy_idx(src_all, dst_small, j):
    for t in range(K // 16):
        dst_small[pl.ds(t * 16, 16)] = src_all[pl.ds(j * K + t * 16, 16)]


def _sc_counts(src_hbm, dst_hbm, cv_out, ce_out,
               sall, dall, s0, s1, d0, d1, ones_b, zb, cv_s, ce_s,
               sem0, sem1):
    c, s, base = _wid_base()
    pltpu.sync_copy(src_hbm.at[pl.ds(base, PAIRS_W)], sall)
    pltpu.sync_copy(dst_hbm.at[pl.ds(base, PAIRS_W)], dall)
    _zero_vec(zb, TV)

    def fill(i, _):
        ones_b[pl.ds(i * 16, 16)] = jnp.ones((16,), jnp.float32)
        return 0
    lax.fori_loop(0, K // 16, fill, 0)
    pltpu.sync_copy(zb, cv_s.at[pl.ds(s * TV, TV)])
    pltpu.sync_copy(zb.at[pl.ds(0, TE)], ce_s.at[pl.ds(s * TE, TE)])
    plsc.subcore_barrier()

    def issue(j, sb, db, sem):
        _copy_idx(sall, sb, j)
        _copy_idx(dall, db, j)
        pltpu.async_copy(ones_b, cv_s.at[sb], sem, add=True)
        pltpu.async_copy(ones_b, ce_s.at[db], sem, add=True)

    def drain(sb, db, sem):
        pltpu.make_async_copy(ones_b, cv_s.at[sb], sem).wait()
        pltpu.make_async_copy(ones_b, ce_s.at[db], sem).wait()

    issue(0, s0, d0, sem0)
    issue(1, s1, d1, sem1)

    def body(i, _):
        drain(s0, d0, sem0)
        issue(2 * i + 2, s0, d0, sem0)

        @pl.when(i < (NCHUNK - 3) // 2)
        def _():
            drain(s1, d1, sem1)
            issue(2 * i + 3, s1, d1, sem1)
        return 0
    lax.fori_loop(0, (NCHUNK - 1) // 2, body, 0)
    drain(s0, d0, sem0)
    drain(s1, d1, sem1)
    plsc.subcore_barrier()

    @pl.when(s == 0)
    def _():
        pltpu.sync_copy(cv_s, cv_out.at[pl.ds(c * NP, NP)])
        pltpu.sync_copy(ce_s, ce_out.at[pl.ds(c * EP, EP)])


def _zero_rows(rows):
    def zr(j, _):
        for k in range(8):
            rows[j, pl.ds(k * 16, 16)] = jnp.zeros((16,), jnp.float32)
        return 0
    lax.fori_loop(0, K, zr, 0)


def _sc_v2e(h_hbm, src_hbm, dst_hbm, xe_out,
            sall, dall, s0, s1, d0, d1, rows0, rows1, acc,
            semg0, semg1, sems0, sems1):
    c, s, base = _wid_base()
    pltpu.sync_copy(src_hbm.at[pl.ds(base, PAIRS_W)], sall)
    pltpu.sync_copy(dst_hbm.at[pl.ds(base, PAIRS_W)], dall)
    _zero_rows(rows0)
    for j in range(TE // K):
        pltpu.sync_copy(rows0, acc.at[pl.ds(s * TE + j * K, K)])
    plsc.subcore_barrier()

    def stage(j, sb, db, rows, semg):
        _copy_idx(sall, sb, j)
        _copy_idx(dall, db, j)
        pltpu.async_copy(h_hbm.at[sb], rows, semg)

    def finish(sb, db, rows, semg):
        pltpu.make_async_copy(h_hbm.at[sb], rows, semg).wait()
        pltpu.sync_copy(rows, acc.at[db], add=True)

    stage(0, s0, d0, rows0, semg0)

    def body(i, _):
        stage(2 * i + 1, s1, d1, rows1, semg1)
        finish(s0, d0, rows0, semg0)
        stage(2 * i + 2, s0, d0, rows0, semg0)
        finish(s1, d1, rows1, semg1)
        return 0
    lax.fori_loop(0, (NCHUNK - 1) // 2, body, 0)
    finish(s0, d0, rows0, semg0)
    plsc.subcore_barrier()
    pltpu.sync_copy(acc.at[pl.ds(s * TE, TE)], xe_out.at[c, pl.ds(s * TE, TE)])


def _sc_stats(src_hbm, dst_hbm, sv_hbm, qv_hbm, se_hbm, qe_hbm,
              rall_out, r_out, mr_out,
              sall, dall, scb0, scb1, rbuf0, rbuf1, mbuf0, mbuf1, rtile,
              tsv, tqv, tse, tqe, racc, macc, sem0, sem1):
    c, s, base = _wid_base()
    pltpu.sync_copy(src_hbm.at[pl.ds(base, PAIRS_W)], sall)
    pltpu.sync_copy(dst_hbm.at[pl.ds(base, PAIRS_W)], dall)
    pltpu.sync_copy(sv_hbm, tsv)
    pltpu.sync_copy(qv_hbm, tqv)
    pltpu.sync_copy(se_hbm, tse)
    pltpu.sync_copy(qe_hbm, tqe)
    _zero_vec(rbuf0, K)
    for j in range(TV // K):
        pltpu.sync_copy(rbuf0, racc.at[pl.ds(s * TV + j * K, K)])
        pltpu.sync_copy(rbuf0, macc.at[pl.ds(s * TV + j * K, K)])
    plsc.subcore_barrier()

    def issue(i, scb, rbuf, mbuf, sem):
        for t in range(K // 16):
            sl = pl.ds(t * 16, 16)
            si = sall[pl.ds(i * K + t * 16, 16)]
            di = dall[pl.ds(i * K + t * 16, 16)]
            scb[sl] = si
            m = plsc.load_gather(tsv, [si]) + plsc.load_gather(tse, [di])
            w = (plsc.load_gather(tqv, [si]) + plsc.load_gather(tqe, [di])
                 - m * m + EPS)
            r = _rsqrt16(w)
            rbuf[sl] = r
            mbuf[sl] = m * r
            rtile[pl.ds(i * K + t * 16, 16)] = r
        pltpu.async_copy(rbuf, racc.at[scb], sem, add=True)
        pltpu.async_copy(mbuf, macc.at[scb], sem, add=True)

    def drain(scb, rbuf, mbuf, sem):
        pltpu.make_async_copy(rbuf, racc.at[scb], sem).wait()
        pltpu.make_async_copy(mbuf, macc.at[scb], sem).wait()

    issue(0, scb0, rbuf0, mbuf0, sem0)
    issue(1, scb1, rbuf1, mbuf1, sem1)

    def body(i, _):
        drain(scb0, rbuf0, mbuf0, sem0)
        issue(2 * i + 2, scb0, rbuf0, mbuf0, sem0)

        @pl.when(i < (NCHUNK - 3) // 2)
        def _():
            drain(scb1, rbuf1, mbuf1, sem1)
            issue(2 * i + 3, scb1, rbuf1, mbuf1, sem1)
        return 0
    lax.fori_loop(0, (NCHUNK - 1) // 2, body, 0)
    drain(scb0, rbuf0, mbuf0, sem0)
    drain(scb1, rbuf1, mbuf1, sem1)
    plsc.subcore_barrier()
    pltpu.sync_copy(rtile, rall_out.at[pl.ds(base, PAIRS_W)])

    @pl.when(s == 0)
    def _():
        pltpu.sync_copy(racc, r_out.at[pl.ds(c * NP, NP)])
        pltpu.sync_copy(macc, mr_out.at[pl.ds(c * NP, NP)])


def _sc_apply(b_hbm, src_hbm, dst_hbm, rall_hbm, sb_out,
              sall, dall, s0, s1, d0, d1, rc0, rc1, rows0, rows1,
              sbacc, semg0, semg1, sems0, sems1):
    c, s, base = _wid_base()
    pltpu.sync_copy(src_hbm.at[pl.ds(base, PAIRS_W)], sall)
    pltpu.sync_copy(dst_hbm.at[pl.ds(base, PAIRS_W)], dall)
    _zero_rows(rows0)
    for j in range(TV // K):
        pltpu.sync_copy(rows0, sbacc.at[pl.ds(s * TV + j * K, K)])
    plsc.subcore_barrier()

    def stage(j, sb, db, rc, rows, semg):
        _copy_idx(sall, sb, j)
        _copy_idx(dall, db, j)
        pltpu.async_copy(b_hbm.at[db], rows, semg)
        pltpu.async_copy(rall_hbm.at[pl.ds(base + j * K, K)], rc, semg)

    def finish(j, sb, db, rc, rows, semg):
        pltpu.make_async_copy(b_hbm.at[db], rows, semg).wait()
        pltpu.make_async_copy(
            rall_hbm.at[pl.ds(base + j * K, K)], rc, semg).wait()

        def rowfn(jj, _):
            j4 = jj * 4
            for u in range(4):
                rj = plsc.load_gather(rc, [jnp.full((16,), j4 + u, jnp.int32)])
                for k in range(8):
                    sl = pl.ds(k * 16, 16)
                    rows[j4 + u, sl] = rows[j4 + u, sl] * rj
            return 0
        lax.fori_loop(0, K // 4, rowfn, 0)
        pltpu.sync_copy(rows, sbacc.at[sb], add=True)

    stage(0, s0, d0, rc0, rows0, semg0)

    def body(i, _):
        stage(2 * i + 1, s1, d1, rc1, rows1, semg1)
        finish(2 * i, s0, d0, rc0, rows0, semg0)
        stage(2 * i + 2, s0, d0, rc0, rows0, semg0)
        finish(2 * i + 1, s1, d1, rc1, rows1, semg1)
        return 0
    lax.fori_loop(0, (NCHUNK - 1) // 2, body, 0)
    finish(NCHUNK - 1, s0, d0, rc0, rows0, semg0)
    plsc.subcore_barrier()
    pltpu.sync_copy(sbacc.at[pl.ds(s * TV, TV)], sb_out.at[c, pl.ds(s * TV, TV)])


_IVEC = pltpu.VMEM((PAIRS_W,), jnp.int32)
_IK = pltpu.VMEM((K,), jnp.int32)
_FK = pltpu.VMEM((K,), jnp.float32)
_ROWS = pltpu.VMEM((K, C), jnp.float32)

_counts_call = functools.partial(
    pl.kernel, _sc_counts, mesh=_SC_MESH, compiler_params=_SC_PARAMS,
    out_type=[jax.ShapeDtypeStruct((NC * NP,), jnp.float32),
              jax.ShapeDtypeStruct((NC * EP,), jnp.float32)],
    scratch_types=[_IVEC, _IVEC, _IK, _IK, _IK, _IK, _FK,
                   pltpu.VMEM((TV,), jnp.float32),
                   pltpu.VMEM_SHARED((NP,), jnp.float32),
                   pltpu.VMEM_SHARED((EP,), jnp.float32),
                   pltpu.SemaphoreType.DMA, pltpu.SemaphoreType.DMA])

_v2e_call = functools.partial(
    pl.kernel, _sc_v2e, mesh=_SC_MESH, compiler_params=_SC_PARAMS,
    out_type=[jax.ShapeDtypeStruct((NC, EP, C), jnp.float32)],
    scratch_types=[_IVEC, _IVEC, _IK, _IK, _IK, _IK, _ROWS, _ROWS,
                   pltpu.VMEM_SHARED((EP, C), jnp.float32),
                   pltpu.SemaphoreType.DMA, pltpu.SemaphoreType.DMA,
                   pltpu.SemaphoreType.DMA, pltpu.SemaphoreType.DMA])

_stats_call = functools.partial(
    pl.kernel, _sc_stats, mesh=_SC_MESH, compiler_params=_SC_PARAMS,
    out_type=[jax.ShapeDtypeStruct((NNZ,), jnp.float32),
              jax.ShapeDtypeStruct((NC * NP,), jnp.float32),
              jax.ShapeDtypeStruct((NC * NP,), jnp.float32)],
    scratch_types=[_IVEC, _IVEC, _IK, _IK, _FK, _FK, _FK, _FK,
                   pltpu.VMEM((PAIRS_W,), jnp.float32),
                   pltpu.VMEM((NP,), jnp.float32), pltpu.VMEM((NP,), jnp.float32),
                   pltpu.VMEM((EP,), jnp.float32), pltpu.VMEM((EP,), jnp.float32),
                   pltpu.VMEM_SHARED((NP,), jnp.float32),
                   pltpu.VMEM_SHARED((NP,), jnp.float32),
                   pltpu.SemaphoreType.DMA, pltpu.SemaphoreType.DMA])

_apply_call = functools.partial(
    pl.kernel, _sc_apply, mesh=_SC_MESH, compiler_params=_SC_PARAMS,
    out_type=[jax.ShapeDtypeStruct((NC, NP, C), jnp.float32)],
    scratch_types=[_IVEC, _IVEC, _IK, _IK, _IK, _IK, _FK, _FK, _ROWS, _ROWS,
                   pltpu.VMEM_SHARED((NP, C), jnp.float32),
                   pltpu.SemaphoreType.DMA, pltpu.SemaphoreType.DMA,
                   pltpu.SemaphoreType.DMA, pltpu.SemaphoreType.DMA])


def _row_spec(blk):
    return pl.BlockSpec((blk, C), lambda i: (i, 0))


def _full(shape):
    return pl.BlockSpec(shape, lambda i: tuple(0 for _ in shape))


def _pad1(x, n):
    return jnp.pad(jnp.reshape(x, (-1,)), (0, n - x.shape[0]))


def kernel(X, src, dst, all_batch, lin_in_w, lin_in_b, w1_ln0_g, w1_ln0_b,
           w1_lin0_w, w1_lin0_b, w1_ln1_g, w1_ln1_b, w1_lin1_w, w1_lin1_b,
           w2_ln_g, w2_ln_b, w2_lin_w, w2_lin_b, w3_ln_g, w3_ln_b, w3_lin_w,
           w3_lin_b, clf_lin0_w, clf_lin0_b, clf_ln_g, clf_ln_b, clf_lin1_w,
           clf_lin1_b):
    f32 = jnp.float32
    row = lambda v: jnp.reshape(v, (1, -1))

    x0 = pl.pallas_call(
        _k_input,
        grid=(GRID,),
        in_specs=[_row_spec(ROWB), _full((C, C)), _full((1, C))],
        out_specs=_row_spec(ROWB),
        out_shape=jax.ShapeDtypeStruct((N, C), f32),
    )(X, lin_in_w, row(lin_in_b))

    gb = jnp.stack([w2_ln_g, w2_ln_b], axis=0)
    gwc = pl.pallas_call(
        _k_prep,
        grid=(1,),
        in_specs=[_full((2, 2 * C)), _full((2 * C, C)), _full((1, C))],
        out_specs=_full((2, C)),
        out_shape=jax.ShapeDtypeStruct((2, C), f32),
    )(gb, w2_lin_w, row(w2_lin_b))

    cv_p, ce_p = _counts_call()(src, dst)
    cv3 = jnp.reshape(cv_p, (NC, NP))[:, :N, None]
    ce3 = jnp.reshape(ce_p, (NC, EP))[:, :EH, None]

    w2top = w2_lin_w[:C]
    w2bot = w2_lin_w[C:]
    g2lo = row(w2_ln_g[:C])
    g2hi = row(w2_ln_g[C:])

    x = x0
    for _ in range(NLAYER):
        h2, a_mat, sv, qv = pl.pallas_call(
            _k_layer_a,
            grid=(GRID,),
            in_specs=[_row_spec(ROWB)] + [_full((1, C))] * 2
            + [_full((C, C)), _full((1, C))] + [_full((1, C))] * 2
            + [_full((C, C)), _full((1, C))] + [_full((1, C)), _full((C, C))],
            out_specs=[_row_spec(ROWB), _row_spec(ROWB),
                       pl.BlockSpec((ROWB, 1), lambda i: (i, 0)),
                       pl.BlockSpec((ROWB, 1), lambda i: (i, 0))],
            out_shape=[jax.ShapeDtypeStruct((N, C), f32),
                       jax.ShapeDtypeStruct((N, C), f32),
                       jax.ShapeDtypeStruct((N, 1), f32),
                       jax.ShapeDtypeStruct((N, 1), f32)],
        )(x, row(w1_ln0_g), row(w1_ln0_b), w1_lin0_w, row(w1_lin0_b),
          row(w1_ln1_g), row(w1_ln1_b), w1_lin1_w, row(w1_lin1_b),
          g2lo, w2top)

        (xe_p,) = _v2e_call()(h2, src, dst)

        b_mat, se, qe = pl.pallas_call(
            _k_edge,
            grid=(EGRID,),
            in_specs=[pl.BlockSpec((NC, EROWB, C), lambda i: (0, i, 0)),
                      pl.BlockSpec((NC, EROWB, 1), lambda i: (0, i, 0)),
                      _full((1, C)), _full((C, C))],
            out_specs=[_row_spec(EROWB),
                       pl.BlockSpec((EROWB, 1), lambda i: (i, 0)),
                       pl.BlockSpec((EROWB, 1), lambda i: (i, 0))],
            out_shape=[jax.ShapeDtypeStruct((EH, C), f32),
                       jax.ShapeDtypeStruct((EH, 1), f32),
                       jax.ShapeDtypeStruct((EH, 1), f32)],
        )(xe_p[:, :EH], ce3, g2hi, w2bot)

        rall, r_p, mr_p = _stats_call()(
            src, dst, _pad1(sv, NP), _pad1(qv, NP),
            _pad1(se, EP), _pad1(qe, EP))
        (sb_p,) = _apply_call()(b_mat, src, dst, rall)

        x = pl.pallas_call(
            _k_vertex,
            grid=(GRID,),
            in_specs=[pl.BlockSpec((NC, ROWB, C), lambda i: (0, i, 0)),
                      pl.BlockSpec((NC, ROWB, 1), lambda i: (0, i, 0)),
                      pl.BlockSpec((NC, ROWB, 1), lambda i: (0, i, 0)),
                      pl.BlockSpec((NC, ROWB, 1), lambda i: (0, i, 0)),
                      _row_spec(ROWB), _row_spec(ROWB), _full((2, C)),
                      _full((1, C)), _full((1, C)), _full((C, C)), _full((1, C))],
            out_specs=_row_spec(ROWB),
            out_shape=jax.ShapeDtypeStruct((N, C), f32),
        )(sb_p[:, :N], jnp.reshape(r_p, (NC, NP))[:, :N, None],
          jnp.reshape(mr_p, (NC, NP))[:, :N, None], cv3, a_mat, x0, gwc,
          row(w3_ln_g), row(w3_ln_b), w3_lin_w, row(w3_lin_b))

    wc1p = jnp.pad(clf_lin1_w, ((0, 0), (0, C - NCLS)))
    bc1p = row(jnp.pad(clf_lin1_b, (0, C - NCLS)))
    ab3 = jnp.reshape(all_batch.astype(jnp.int32), (GRID, 1, ROWB))
    readout = pl.pallas_call(
        _k_clf,
        grid=(GRID,),
        in_specs=[_row_spec(ROWB), _full((C, C)), _full((1, C)),
                  _full((1, C)), _full((1, C)), _full((C, C)), _full((1, C)),
                  pl.BlockSpec((1, 1, ROWB), lambda i: (i, 0, 0))],
        out_specs=_full((NGRAPH, NCLS)),
        out_shape=jax.ShapeDtypeStruct((NGRAPH, NCLS), f32),
        scratch_shapes=[pltpu.VMEM((NGRAPH, C), f32),
                        pltpu.VMEM((NGRAPH, C), f32)],
    )(x, clf_lin0_w, row(clf_lin0_b), row(clf_ln_g), row(clf_ln_b),
      wc1p, bc1p, ab3)
    return readout


# speedup vs baseline: 1.1577x; 1.0156x over previous
import functools

import jax
import jax.numpy as jnp
from jax import lax
from jax.experimental import pallas as pl
from jax.experimental.pallas import tpu as pltpu
from jax.experimental.pallas import tpu_sc as plsc

N, EH, NNZ, C, NCLS, NGRAPH, NLAYER, ALPHA = 10000, 5000, 320000, 128, 10, 16, 2, 0.5
EPS = 1e-5
NP = 10240
EP = 5120
NC, NS = 2, 16
NW = NC * NS
PAIRS_W = NNZ // NW
K = 80
NCHUNK = PAIRS_W // K
TV = NP // NS
TE = EP // NS
ROWB = 2000
GRID = N // ROWB
EROWB = 1000
EGRID = EH // EROWB

_SC_MESH = plsc.VectorSubcoreMesh(core_axis_name="c", subcore_axis_name="s")
_SC_PARAMS = pltpu.CompilerParams(needs_layout_passes=False)


def _ln(x, g, b):
    m = jnp.mean(x, axis=-1, keepdims=True)
    v = jnp.mean((x - m) ** 2, axis=-1, keepdims=True)
    return (x - m) * lax.rsqrt(v + EPS) * g + b


def _dot(a, b):
    return jnp.dot(a, b, preferred_element_type=jnp.float32)


def _k_input(x_ref, w_ref, b_ref, o_ref):
    o_ref[...] = jnp.maximum(_dot(x_ref[...], w_ref[...]) + b_ref[...], 0.0)


def _k_prep(gb_ref, w2_ref, b2_ref, o_ref):
    o = _dot(gb_ref[...], w2_ref[...])
    o_ref[...] = o + jnp.concatenate(
        [jnp.zeros_like(b2_ref[...]), b2_ref[...]], axis=0)


def _k_layer_a(x_ref, g0, b0, w10, b10, g1, b1, w11, b11, g2lo, w2top,
               h2_o, a_o, sv_o, qv_o):
    x = x_ref[...]
    h = _ln(x, g0[...], b0[...])
    h = jnp.maximum(_dot(h, w10[...]) + b10[...], 0.0)
    h = _ln(h, g1[...], b1[...])
    h2_o[...] = _dot(h, w11[...]) + b11[...]
    a_o[...] = _dot(x * g2lo[...], w2top[...])
    sv_o[...] = jnp.sum(x, axis=-1, keepdims=True) * (1.0 / (2 * C))
    qv_o[...] = jnp.sum(x * x, axis=-1, keepdims=True) * (1.0 / (2 * C))


def _k_edge(xs_ref, ce_ref, g2hi, w2bot, b_o, se_o, qe_o):
    xs = xs_ref[...]
    cnt = ce_ref[0] + ce_ref[1]
    xe = (xs[0] + xs[1]) / jnp.clip(cnt, 1.0, None)
    b_o[...] = _dot(xe * g2hi[...], w2bot[...])
    se_o[...] = jnp.sum(xe, axis=-1, keepdims=True) * (1.0 / (2 * C))
    qe_o[...] = jnp.sum(xe * xe, axis=-1, keepdims=True) * (1.0 / (2 * C))


def _k_vertex(sb_ref, r_ref, mr_ref, cv_ref, a_ref, x0_ref, gwc_ref,
              g3, b3, w3, b3l, x_o):
    cnt = cv_ref[0] + cv_ref[1]
    gw = gwc_ref[0:1]
    bwc = gwc_ref[1:2]
    seg = (a_ref[...] * (r_ref[0] + r_ref[1])
           + (sb_ref[0] + sb_ref[1])
           - (mr_ref[0] + mr_ref[1]) * gw
           + cnt * bwc)
    xv = seg / jnp.clip(cnt, 1.0, None)
    xn = (1.0 - ALPHA) * xv + ALPHA * x0_ref[...]
    x_o[...] = jnp.maximum(_dot(_ln(xn, g3[...], b3[...]), w3[...]) + b3l[...], 0.0)


def _k_clf(x_ref, wc0, bc0, gc, bc, wc1, bc1, ab_ref, out_ref, sum_scr, cnt_scr):
    i = pl.program_id(0)

    @pl.when(i == 0)
    def _():
        sum_scr[...] = jnp.zeros_like(sum_scr)
        cnt_scr[...] = jnp.zeros_like(cnt_scr)

    h = jnp.maximum(_dot(x_ref[...], wc0[...]) + bc0[...], 0.0)
    h = _ln(h, gc[...], bc[...])
    o = _dot(h, wc1[...]) + bc1[...]
    ab = ab_ref[0]
    gids = lax.broadcasted_iota(jnp.int32, (NGRAPH, ROWB), 0)
    oh = jnp.where(ab == gids, 1.0, 0.0)
    sum_scr[...] += _dot(oh, o)
    cnt_scr[...] += jnp.broadcast_to(
        jnp.sum(oh, axis=-1, keepdims=True), (NGRAPH, C))

    @pl.when(i == GRID - 1)
    def _():
        out_ref[...] = (sum_scr[...] / jnp.clip(cnt_scr[...], 1.0, None))[:, :NCLS]


def _zero_vec(ref, n):
    def body(i, _):
        ref[pl.ds(i * 16, 16)] = jnp.zeros((16,), jnp.float32)
        return 0
    lax.fori_loop(0, n // 16, body, 0)


def _rsqrt16(w):
    i = plsc.bitcast(w, jnp.int32)
    i = 0x5F3759DF - lax.shift_right_logical(i, 1)
    y = plsc.bitcast(i, jnp.float32)
    for _ in range(4):
        y = y * (1.5 - 0.5 * w * y * y)
    return y


def _wid_base():
    c = lax.axis_index("c")
    s = lax.axis_index("s")
    return c, s, (s * NC + c) * PAIRS_W


def _copy_idx(src_all, dst_small, j):
    for t in range(K // 16):
        dst_small[pl.ds(t * 16, 16)] = src_all[pl.ds(j * K + t * 16, 16)]


def _sc_counts(src_hbm, dst_hbm, cv_out, ce_out,
               sall, dall, s0, s1, d0, d1, ones_b, zb, cv_s, ce_s,
               sem0, sem1):
    c, s, base = _wid_base()
    pltpu.sync_copy(src_hbm.at[pl.ds(base, PAIRS_W)], sall)
    pltpu.sync_copy(dst_hbm.at[pl.ds(base, PAIRS_W)], dall)
    _zero_vec(zb, TV)

    def fill(i, _):
        ones_b[pl.ds(i * 16, 16)] = jnp.ones((16,), jnp.float32)
        return 0
    lax.fori_loop(0, K // 16, fill, 0)
    pltpu.sync_copy(zb, cv_s.at[pl.ds(s * TV, TV)])
    pltpu.sync_copy(zb.at[pl.ds(0, TE)], ce_s.at[pl.ds(s * TE, TE)])
    plsc.subcore_barrier()

    def issue(j, sb, db, sem):
        _copy_idx(sall, sb, j)
        _copy_idx(dall, db, j)
        pltpu.async_copy(ones_b, cv_s.at[sb], sem, add=True)
        pltpu.async_copy(ones_b, ce_s.at[db], sem, add=True)

    def drain(sb, db, sem):
        pltpu.make_async_copy(ones_b, cv_s.at[sb], sem).wait()
        pltpu.make_async_copy(ones_b, ce_s.at[db], sem).wait()

    issue(0, s0, d0, sem0)
    issue(1, s1, d1, sem1)

    def body(i, _):
        drain(s0, d0, sem0)
        issue(2 * i + 2, s0, d0, sem0)

        @pl.when(i < (NCHUNK - 3) // 2)
        def _():
            drain(s1, d1, sem1)
            issue(2 * i + 3, s1, d1, sem1)
        return 0
    lax.fori_loop(0, (NCHUNK - 1) // 2, body, 0)
    drain(s0, d0, sem0)
    drain(s1, d1, sem1)
    plsc.subcore_barrier()

    @pl.when(s == 0)
    def _():
        pltpu.sync_copy(cv_s, cv_out.at[pl.ds(c * NP, NP)])
        pltpu.sync_copy(ce_s, ce_out.at[pl.ds(c * EP, EP)])


def _zero_rows(rows):
    def zr(j, _):
        for k in range(8):
            rows[j, pl.ds(k * 16, 16)] = jnp.zeros((16,), jnp.float32)
        return 0
    lax.fori_loop(0, K, zr, 0)


def _sc_v2e(h_hbm, src_hbm, dst_hbm, xe_out,
            sall, dall, s0, s1, d0, d1, rows0, rows1, acc,
            semg0, semg1, sems0, sems1):
    c, s, base = _wid_base()
    pltpu.sync_copy(src_hbm.at[pl.ds(base, PAIRS_W)], sall)
    pltpu.sync_copy(dst_hbm.at[pl.ds(base, PAIRS_W)], dall)
    _zero_rows(rows0)
    for j in range(TE // K):
        pltpu.sync_copy(rows0, acc.at[pl.ds(s * TE + j * K, K)])
    plsc.subcore_barrier()

    def stage(j, sb, db, rows, semg):
        _copy_idx(sall, sb, j)
        _copy_idx(dall, db, j)
        pltpu.async_copy(h_hbm.at[sb], rows, semg)

    def finish(sb, db, rows, semg):
        pltpu.make_async_copy(h_hbm.at[sb], rows, semg).wait()
        pltpu.sync_copy(rows, acc.at[db], add=True)

    stage(0, s0, d0, rows0, semg0)

    def body(i, _):
        stage(2 * i + 1, s1, d1, rows1, semg1)
        finish(s0, d0, rows0, semg0)
        stage(2 * i + 2, s0, d0, rows0, semg0)
        finish(s1, d1, rows1, semg1)
        return 0
    lax.fori_loop(0, (NCHUNK - 1) // 2, body, 0)
    finish(s0, d0, rows0, semg0)
    plsc.subcore_barrier()
    pltpu.sync_copy(acc.at[pl.ds(s * TE, TE)], xe_out.at[c, pl.ds(s * TE, TE)])


def _sc_e2v(b_hbm, src_hbm, dst_hbm, sv_hbm, qv_hbm, se_hbm, qe_hbm,
            sb_out, r_out, mr_out,
            sall, dall, s0, s1, d0, d1, scb0, scb1,
            tv0, tv1, tq0, tq1, te0, te1, tqe0, tqe1,
            rb0, rb1, mb0, mb1, rows0, rows1,
            sbacc, racc, macc, semg0, semg1, sema0, sema1):
    c, s, base = _wid_base()
    pltpu.sync_copy(src_hbm.at[pl.ds(base, PAIRS_W)], sall)
    pltpu.sync_copy(dst_hbm.at[pl.ds(base, PAIRS_W)], dall)
    _zero_rows(rows0)
    _zero_vec(rb0, K)
    _zero_vec(rb1, K)
    _zero_vec(mb0, K)
    _zero_vec(mb1, K)

    def zi(i, _):
        scb0[pl.ds(i * 16, 16)] = jnp.zeros((16,), jnp.int32)
        scb1[pl.ds(i * 16, 16)] = jnp.zeros((16,), jnp.int32)
        return 0
    lax.fori_loop(0, K // 16, zi, 0)
    for j in range(TV // K):
        pltpu.sync_copy(rows0, sbacc.at[pl.ds(s * TV + j * K, K)])
        pltpu.sync_copy(rb0, racc.at[pl.ds(s * TV + j * K, K)])
        pltpu.sync_copy(rb0, macc.at[pl.ds(s * TV + j * K, K)])
    plsc.subcore_barrier()
    pltpu.async_copy(rb0, racc.at[scb0], sema0, add=True)
    pltpu.async_copy(mb0, macc.at[scb0], sema0, add=True)
    pltpu.async_copy(rb1, racc.at[scb1], sema1, add=True)
    pltpu.async_copy(mb1, macc.at[scb1], sema1, add=True)

    def stage(j, sb, db, tv, tq, te, tqe, rows, semg):
        _copy_idx(sall, sb, j)
        _copy_idx(dall, db, j)
        pltpu.async_copy(b_hbm.at[db], rows, semg)
        pltpu.async_copy(sv_hbm.at[sb], tv, semg)
        pltpu.async_copy(qv_hbm.at[sb], tq, semg)
        pltpu.async_copy(se_hbm.at[db], te, semg)
        pltpu.async_copy(qe_hbm.at[db], tqe, semg)

    def finish(sb, db, scb, tv, tq, te, tqe, rb, mb, rows, semg, sema):
        pltpu.make_async_copy(rb, racc.at[scb], sema).wait()
        pltpu.make_async_copy(mb, macc.at[scb], sema).wait()
        pltpu.make_async_copy(b_hbm.at[db], rows, semg).wait()
        pltpu.make_async_copy(sv_hbm.at[sb], tv, semg).wait()
        pltpu.make_async_copy(qv_hbm.at[sb], tq, semg).wait()
        pltpu.make_async_copy(se_hbm.at[db], te, semg).wait()
        pltpu.make_async_copy(qe_hbm.at[db], tqe, semg).wait()
        for t in range(K // 16):
            sl = pl.ds(t * 16, 16)
            scb[sl] = sb[sl]
            m = tv[sl] + te[sl]
            w = tq[sl] + tqe[sl] - m * m + EPS
            r = _rsqrt16(w)
            rb[sl] = r
            mb[sl] = m * r
        pltpu.async_copy(rb, racc.at[scb], sema, add=True)
        pltpu.async_copy(mb, macc.at[scb], sema, add=True)

        def rowfn(jj, _):
            j4 = jj * 4
            for u in range(4):
                rj = plsc.load_gather(rb, [jnp.full((16,), j4 + u, jnp.int32)])
                for k in range(8):
                    sl = pl.ds(k * 16, 16)
                    rows[j4 + u, sl] = rows[j4 + u, sl] * rj
            return 0
        lax.fori_loop(0, K // 4, rowfn, 0)
        pltpu.sync_copy(rows, sbacc.at[sb], add=True)

    def fin0():
        finish(s0, d0, scb0, tv0, tq0, te0, tqe0, rb0, mb0, rows0,
               semg0, sema0)

    def fin1():
        finish(s1, d1, scb1, tv1, tq1, te1, tqe1, rb1, mb1, rows1,
               semg1, sema1)

    stage(0, s0, d0, tv0, tq0, te0, tqe0, rows0, semg0)

    def body(i, _):
        stage(2 * i + 1, s1, d1, tv1, tq1, te1, tqe1, rows1, semg1)
        fin0()
        stage(2 * i + 2, s0, d0, tv0, tq0, te0, tqe0, rows0, semg0)
        fin1()
        return 0
    lax.fori_loop(0, (NCHUNK - 1) // 2, body, 0)
    fin0()
    pltpu.make_async_copy(rb0, racc.at[scb0], sema0).wait()
    pltpu.make_async_copy(mb0, macc.at[scb0], sema0).wait()
    pltpu.make_async_copy(rb1, racc.at[scb1], sema1).wait()
    pltpu.make_async_copy(mb1, macc.at[scb1], sema1).wait()
    plsc.subcore_barrier()
    pltpu.sync_copy(sbacc.at[pl.ds(s * TV, TV)], sb_out.at[c, pl.ds(s * TV, TV)])

    @pl.when(s == 0)
    def _():
        pltpu.sync_copy(racc, r_out.at[pl.ds(c * NP, NP)])
        pltpu.sync_copy(macc, mr_out.at[pl.ds(c * NP, NP)])


_IVEC = pltpu.VMEM((PAIRS_W,), jnp.int32)
_IK = pltpu.VMEM((K,), jnp.int32)
_FK = pltpu.VMEM((K,), jnp.float32)
_ROWS = pltpu.VMEM((K, C), jnp.float32)

_counts_call = functools.partial(
    pl.kernel, _sc_counts, mesh=_SC_MESH, compiler_params=_SC_PARAMS,
    out_type=[jax.ShapeDtypeStruct((NC * NP,), jnp.float32),
              jax.ShapeDtypeStruct((NC * EP,), jnp.float32)],
    scratch_types=[_IVEC, _IVEC, _IK, _IK, _IK, _IK, _FK,
                   pltpu.VMEM((TV,), jnp.float32),
                   pltpu.VMEM_SHARED((NP,), jnp.float32),
                   pltpu.VMEM_SHARED((EP,), jnp.float32),
                   pltpu.SemaphoreType.DMA, pltpu.SemaphoreType.DMA])

_v2e_call = functools.partial(
    pl.kernel, _sc_v2e, mesh=_SC_MESH, compiler_params=_SC_PARAMS,
    out_type=[jax.ShapeDtypeStruct((NC, EP, C), jnp.float32)],
    scratch_types=[_IVEC, _IVEC, _IK, _IK, _IK, _IK, _ROWS, _ROWS,
                   pltpu.VMEM_SHARED((EP, C), jnp.float32),
                   pltpu.SemaphoreType.DMA, pltpu.SemaphoreType.DMA,
                   pltpu.SemaphoreType.DMA, pltpu.SemaphoreType.DMA])

_e2v_call = functools.partial(
    pl.kernel, _sc_e2v, mesh=_SC_MESH, compiler_params=_SC_PARAMS,
    out_type=[jax.ShapeDtypeStruct((NC, NP, C), jnp.float32),
              jax.ShapeDtypeStruct((NC * NP,), jnp.float32),
              jax.ShapeDtypeStruct((NC * NP,), jnp.float32)],
    scratch_types=[_IVEC, _IVEC, _IK, _IK, _IK, _IK, _IK, _IK,
                   _FK, _FK, _FK, _FK, _FK, _FK, _FK, _FK,
                   _FK, _FK, _FK, _FK, _ROWS, _ROWS,
                   pltpu.VMEM_SHARED((NP, C), jnp.float32),
                   pltpu.VMEM_SHARED((NP,), jnp.float32),
                   pltpu.VMEM_SHARED((NP,), jnp.float32),
                   pltpu.SemaphoreType.DMA, pltpu.SemaphoreType.DMA,
                   pltpu.SemaphoreType.DMA, pltpu.SemaphoreType.DMA])


def _row_spec(blk):
    return pl.BlockSpec((blk, C), lambda i: (i, 0))


def _full(shape):
    return pl.BlockSpec(shape, lambda i: tuple(0 for _ in shape))


def _pad1(x, n):
    return jnp.pad(jnp.reshape(x, (-1,)), (0, n - x.shape[0]))


def kernel(X, src, dst, all_batch, lin_in_w, lin_in_b, w1_ln0_g, w1_ln0_b,
           w1_lin0_w, w1_lin0_b, w1_ln1_g, w1_ln1_b, w1_lin1_w, w1_lin1_b,
           w2_ln_g, w2_ln_b, w2_lin_w, w2_lin_b, w3_ln_g, w3_ln_b, w3_lin_w,
           w3_lin_b, clf_lin0_w, clf_lin0_b, clf_ln_g, clf_ln_b, clf_lin1_w,
           clf_lin1_b):
    f32 = jnp.float32
    row = lambda v: jnp.reshape(v, (1, -1))

    x0 = pl.pallas_call(
        _k_input,
        grid=(GRID,),
        in_specs=[_row_spec(ROWB), _full((C, C)), _full((1, C))],
        out_specs=_row_spec(ROWB),
        out_shape=jax.ShapeDtypeStruct((N, C), f32),
    )(X, lin_in_w, row(lin_in_b))

    gb = jnp.stack([w2_ln_g, w2_ln_b], axis=0)
    gwc = pl.pallas_call(
        _k_prep,
        grid=(1,),
        in_specs=[_full((2, 2 * C)), _full((2 * C, C)), _full((1, C))],
        out_specs=_full((2, C)),
        out_shape=jax.ShapeDtypeStruct((2, C), f32),
    )(gb, w2_lin_w, row(w2_lin_b))

    cv_p, ce_p = _counts_call()(src, dst)
    cv3 = jnp.reshape(cv_p, (NC, NP))[:, :N, None]
    ce3 = jnp.reshape(ce_p, (NC, EP))[:, :EH, None]

    w2top = w2_lin_w[:C]
    w2bot = w2_lin_w[C:]
    g2lo = row(w2_ln_g[:C])
    g2hi = row(w2_ln_g[C:])

    x = x0
    for _ in range(NLAYER):
        h2, a_mat, sv, qv = pl.pallas_call(
            _k_layer_a,
            grid=(GRID,),
            in_specs=[_row_spec(ROWB)] + [_full((1, C))] * 2
            + [_full((C, C)), _full((1, C))] + [_full((1, C))] * 2
            + [_full((C, C)), _full((1, C))] + [_full((1, C)), _full((C, C))],
            out_specs=[_row_spec(ROWB), _row_spec(ROWB),
                       pl.BlockSpec((ROWB, 1), lambda i: (i, 0)),
                       pl.BlockSpec((ROWB, 1), lambda i: (i, 0))],
            out_shape=[jax.ShapeDtypeStruct((N, C), f32),
                       jax.ShapeDtypeStruct((N, C), f32),
                       jax.ShapeDtypeStruct((N, 1), f32),
                       jax.ShapeDtypeStruct((N, 1), f32)],
        )(x, row(w1_ln0_g), row(w1_ln0_b), w1_lin0_w, row(w1_lin0_b),
          row(w1_ln1_g), row(w1_ln1_b), w1_lin1_w, row(w1_lin1_b),
          g2lo, w2top)

        (xe_p,) = _v2e_call()(h2, src, dst)

        b_mat, se, qe = pl.pallas_call(
            _k_edge,
            grid=(EGRID,),
            in_specs=[pl.BlockSpec((NC, EROWB, C), lambda i: (0, i, 0)),
                      pl.BlockSpec((NC, EROWB, 1), lambda i: (0, i, 0)),
                      _full((1, C)), _full((C, C))],
            out_specs=[_row_spec(EROWB),
                       pl.BlockSpec((EROWB, 1), lambda i: (i, 0)),
                       pl.BlockSpec((EROWB, 1), lambda i: (i, 0))],
            out_shape=[jax.ShapeDtypeStruct((EH, C), f32),
                       jax.ShapeDtypeStruct((EH, 1), f32),
                       jax.ShapeDtypeStruct((EH, 1), f32)],
        )(xe_p[:, :EH], ce3, g2hi, w2bot)

        sb_p, r_p, mr_p = _e2v_call()(
            b_mat, src, dst, _pad1(sv, NP), _pad1(qv, NP),
            _pad1(se, EP), _pad1(qe, EP))

        x = pl.pallas_call(
            _k_vertex,
            grid=(GRID,),
            in_specs=[pl.BlockSpec((NC, ROWB, C), lambda i: (0, i, 0)),
                      pl.BlockSpec((NC, ROWB, 1), lambda i: (0, i, 0)),
                      pl.BlockSpec((NC, ROWB, 1), lambda i: (0, i, 0)),
                      pl.BlockSpec((NC, ROWB, 1), lambda i: (0, i, 0)),
                      _row_spec(ROWB), _row_spec(ROWB), _full((2, C)),
                      _full((1, C)), _full((1, C)), _full((C, C)), _full((1, C))],
            out_specs=_row_spec(ROWB),
            out_shape=jax.ShapeDtypeStruct((N, C), f32),
        )(sb_p[:, :N], jnp.reshape(r_p, (NC, NP))[:, :N, None],
          jnp.reshape(mr_p, (NC, NP))[:, :N, None], cv3, a_mat, x0, gwc,
          row(w3_ln_g), row(w3_ln_b), w3_lin_w, row(w3_lin_b))

    wc1p = jnp.pad(clf_lin1_w, ((0, 0), (0, C - NCLS)))
    bc1p = row(jnp.pad(clf_lin1_b, (0, C - NCLS)))
    ab3 = jnp.reshape(all_batch.astype(jnp.int32), (GRID, 1, ROWB))
    readout = pl.pallas_call(
        _k_clf,
        grid=(GRID,),
        in_specs=[_row_spec(ROWB), _full((C, C)), _full((1, C)),
                  _full((1, C)), _full((1, C)), _full((C, C)), _full((1, C)),
                  pl.BlockSpec((1, 1, ROWB), lambda i: (i, 0, 0))],
        out_specs=_full((NGRAPH, NCLS)),
        out_shape=jax.ShapeDtypeStruct((NGRAPH, NCLS), f32),
        scratch_shapes=[pltpu.VMEM((NGRAPH, C), f32),
                        pltpu.VMEM((NGRAPH, C), f32)],
    )(x, clf_lin0_w, row(clf_lin0_b), row(clf_ln_g), row(clf_ln_b),
      wc1p, bc1p, ab3)
    return readout


# speedup vs baseline: 1.1717x; 1.0122x over previous
import functools

import jax
import jax.numpy as jnp
from jax import lax
from jax.experimental import pallas as pl
from jax.experimental.pallas import tpu as pltpu
from jax.experimental.pallas import tpu_sc as plsc

N, EH, NNZ, C, NCLS, NGRAPH, NLAYER, ALPHA = 10000, 5000, 320000, 128, 10, 16, 2, 0.5
EPS = 1e-5
NP = 10240
EP = 5120
NC, NS = 2, 16
NW = NC * NS
PAIRS_W = NNZ // NW
K = 80
NCHUNK = PAIRS_W // K
TV = NP // NS
TE = EP // NS
ROWB = 2000
GRID = N // ROWB
EROWB = 1000
EGRID = EH // EROWB

_SC_MESH = plsc.VectorSubcoreMesh(core_axis_name="c", subcore_axis_name="s")
_SC_PARAMS = pltpu.CompilerParams(needs_layout_passes=False)


def _ln(x, g, b):
    m = jnp.mean(x, axis=-1, keepdims=True)
    v = jnp.mean((x - m) ** 2, axis=-1, keepdims=True)
    return (x - m) * lax.rsqrt(v + EPS) * g + b


def _dot(a, b):
    return jnp.dot(a, b, preferred_element_type=jnp.float32)


def _k_prep(gb_ref, w2_ref, b2_ref, o_ref):
    o = _dot(gb_ref[...], w2_ref[...])
    o_ref[...] = o + jnp.concatenate(
        [jnp.zeros_like(b2_ref[...]), b2_ref[...]], axis=0)


def _layer_a(x, g0, b0, w10, b10, g1, b1, w11, b11, g2lo, w2top,
             h2_o, a_o, sv_o, qv_o):
    h = _ln(x, g0[...], b0[...])
    h = jnp.maximum(_dot(h, w10[...]) + b10[...], 0.0)
    h = _ln(h, g1[...], b1[...])
    h2_o[...] = _dot(h, w11[...]) + b11[...]
    a_o[...] = _dot(x * g2lo[...], w2top[...])
    sv_o[...] = jnp.sum(x, axis=-1, keepdims=True) * (1.0 / (2 * C))
    qv_o[...] = jnp.sum(x * x, axis=-1, keepdims=True) * (1.0 / (2 * C))


def _k_in_a(x_ref, wi, bi, g0, b0, w10, b10, g1, b1, w11, b11, g2lo, w2top,
            x0_o, h2_o, a_o, sv_o, qv_o):
    x = jnp.maximum(_dot(x_ref[...], wi[...]) + bi[...], 0.0)
    x0_o[...] = x
    _layer_a(x, g0, b0, w10, b10, g1, b1, w11, b11, g2lo, w2top,
             h2_o, a_o, sv_o, qv_o)


def _vertex(sb_ref, r_ref, mr_ref, cv_ref, a_ref, x0_ref, gwc_ref,
            g3, b3, w3, b3l):
    cnt = cv_ref[0] + cv_ref[1]
    gw = gwc_ref[0:1]
    bwc = gwc_ref[1:2]
    seg = (a_ref[...] * (r_ref[0] + r_ref[1])
           + (sb_ref[0] + sb_ref[1])
           - (mr_ref[0] + mr_ref[1]) * gw
           + cnt * bwc)
    xv = seg / jnp.clip(cnt, 1.0, None)
    xn = (1.0 - ALPHA) * xv + ALPHA * x0_ref[...]
    return jnp.maximum(
        _dot(_ln(xn, g3[...], b3[...]), w3[...]) + b3l[...], 0.0)


def _k_vertex_a(sb_ref, r_ref, mr_ref, cv_ref, a_ref, x0_ref, gwc_ref,
                g3, b3, w3, b3l, g0, b0, w10, b10, g1, b1, w11, b11,
                g2lo, w2top, h2_o, a_o, sv_o, qv_o):
    x = _vertex(sb_ref, r_ref, mr_ref, cv_ref, a_ref, x0_ref, gwc_ref,
                g3, b3, w3, b3l)
    _layer_a(x, g0, b0, w10, b10, g1, b1, w11, b11, g2lo, w2top,
             h2_o, a_o, sv_o, qv_o)


def _k_vertex_clf(sb_ref, r_ref, mr_ref, cv_ref, a_ref, x0_ref, gwc_ref,
                  g3, b3, w3, b3l, wc0, bc0, gc, bc, wc1, bc1, ab_ref,
                  out_ref, sum_scr, cnt_scr):
    i = pl.program_id(0)

    @pl.when(i == 0)
    def _():
        sum_scr[...] = jnp.zeros_like(sum_scr)
        cnt_scr[...] = jnp.zeros_like(cnt_scr)

    x = _vertex(sb_ref, r_ref, mr_ref, cv_ref, a_ref, x0_ref, gwc_ref,
                g3, b3, w3, b3l)
    h = jnp.maximum(_dot(x, wc0[...]) + bc0[...], 0.0)
    h = _ln(h, gc[...], bc[...])
    o = _dot(h, wc1[...]) + bc1[...]
    ab = ab_ref[0]
    gids = lax.broadcasted_iota(jnp.int32, (NGRAPH, ROWB), 0)
    oh = jnp.where(ab == gids, 1.0, 0.0)
    sum_scr[...] += _dot(oh, o)
    cnt_scr[...] += jnp.broadcast_to(
        jnp.sum(oh, axis=-1, keepdims=True), (NGRAPH, C))

    @pl.when(i == GRID - 1)
    def _():
        out_ref[...] = (sum_scr[...] / jnp.clip(cnt_scr[...], 1.0, None))[:, :NCLS]


def _k_edge(xs_ref, ce_ref, g2hi, w2bot, b_o, se_o, qe_o):
    xs = xs_ref[...]
    cnt = ce_ref[0] + ce_ref[1]
    xe = (xs[0] + xs[1]) / jnp.clip(cnt, 1.0, None)
    b_o[...] = _dot(xe * g2hi[...], w2bot[...])
    se_o[...] = jnp.sum(xe, axis=-1, keepdims=True) * (1.0 / (2 * C))
    qe_o[...] = jnp.sum(xe * xe, axis=-1, keepdims=True) * (1.0 / (2 * C))


def _zero_vec(ref, n):
    def body(i, _):
        ref[pl.ds(i * 16, 16)] = jnp.zeros((16,), jnp.float32)
        return 0
    lax.fori_loop(0, n // 16, body, 0)


def _rsqrt16(w):
    i = plsc.bitcast(w, jnp.int32)
    i = 0x5F3759DF - lax.shift_right_logical(i, 1)
    y = plsc.bitcast(i, jnp.float32)
    for _ in range(4):
        y = y * (1.5 - 0.5 * w * y * y)
    return y


def _wid_base():
    c = lax.axis_index("c")
    s = lax.axis_index("s")
    return c, s, (s * NC + c) * PAIRS_W


def _copy_idx(src_all, dst_small, j):
    for t in range(K // 16):
        dst_small[pl.ds(t * 16, 16)] = src_all[pl.ds(j * K + t * 16, 16)]


def _sc_counts(src_hbm, dst_hbm, cv_out, ce_out,
               sall, dall, s0, s1, d0, d1, ones_b, zb, cv_s, ce_s,
               sem0, sem1):
    c, s, base = _wid_base()
    pltpu.sync_copy(src_hbm.at[pl.ds(base, PAIRS_W)], sall)
    pltpu.sync_copy(dst_hbm.at[pl.ds(base, PAIRS_W)], dall)
    _zero_vec(zb, TV)

    def fill(i, _):
        ones_b[pl.ds(i * 16, 16)] = jnp.ones((16,), jnp.float32)
        return 0
    lax.fori_loop(0, K // 16, fill, 0)
    pltpu.sync_copy(zb, cv_s.at[pl.ds(s * TV, TV)])
    pltpu.sync_copy(zb.at[pl.ds(0, TE)], ce_s.at[pl.ds(s * TE, TE)])
    plsc.subcore_barrier()

    def issue(j, sb, db, sem):
        _copy_idx(sall, sb, j)
        _copy_idx(dall, db, j)
        pltpu.async_copy(ones_b, cv_s.at[sb], sem, add=True)
        pltpu.async_copy(ones_b, ce_s.at[db], sem, add=True)

    def drain(sb, db, sem):
        pltpu.make_async_copy(ones_b, cv_s.at[sb], sem).wait()
        pltpu.make_async_copy(ones_b, ce_s.at[db], sem).wait()

    issue(0, s0, d0, sem0)
    issue(1, s1, d1, sem1)

    def body(i, _):
        drain(s0, d0, sem0)
        issue(2 * i + 2, s0, d0, sem0)

        @pl.when(i < (NCHUNK - 3) // 2)
        def _():
            drain(s1, d1, sem1)
            issue(2 * i + 3, s1, d1, sem1)
        return 0
    lax.fori_loop(0, (NCHUNK - 1) // 2, body, 0)
    drain(s0, d0, sem0)
    drain(s1, d1, sem1)
    plsc.subcore_barrier()

    @pl.when(s == 0)
    def _():
        pltpu.sync_copy(cv_s, cv_out.at[pl.ds(c * NP, NP)])
        pltpu.sync_copy(ce_s, ce_out.at[pl.ds(c * EP, EP)])


def _zero_rows(rows):
    def zr(j, _):
        for k in range(8):
            rows[j, pl.ds(k * 16, 16)] = jnp.zeros((16,), jnp.float32)
        return 0
    lax.fori_loop(0, K, zr, 0)


def _sc_v2e(h_hbm, src_hbm, dst_hbm, xe_out,
            sall, dall, s0, s1, d0, d1, rows0, rows1, acc,
            semg0, semg1, sems0, sems1):
    c, s, base = _wid_base()
    pltpu.sync_copy(src_hbm.at[pl.ds(base, PAIRS_W)], sall)
    pltpu.sync_copy(dst_hbm.at[pl.ds(base, PAIRS_W)], dall)
    _zero_rows(rows0)
    for j in range(TE // K):
        pltpu.sync_copy(rows0, acc.at[pl.ds(s * TE + j * K, K)])
    plsc.subcore_barrier()

    def stage(j, sb, db, rows, semg):
        _copy_idx(sall, sb, j)
        _copy_idx(dall, db, j)
        pltpu.async_copy(h_hbm.at[sb], rows, semg)

    def finish(sb, db, rows, semg):
        pltpu.make_async_copy(h_hbm.at[sb], rows, semg).wait()
        pltpu.sync_copy(rows, acc.at[db], add=True)

    stage(0, s0, d0, rows0, semg0)

    def body(i, _):
        stage(2 * i + 1, s1, d1, rows1, semg1)
        finish(s0, d0, rows0, semg0)
        stage(2 * i + 2, s0, d0, rows0, semg0)
        finish(s1, d1, rows1, semg1)
        return 0
    lax.fori_loop(0, (NCHUNK - 1) // 2, body, 0)
    finish(s0, d0, rows0, semg0)
    plsc.subcore_barrier()
    pltpu.sync_copy(acc.at[pl.ds(s * TE, TE)], xe_out.at[c, pl.ds(s * TE, TE)])


def _sc_e2v(b_hbm, src_hbm, dst_hbm, sv_hbm, qv_hbm, se_hbm, qe_hbm,
            sb_out, r_out, mr_out,
            sall, dall, s0, s1, d0, d1, scb0, scb1,
            tv0, tv1, tq0, tq1, te0, te1, tqe0, tqe1,
            rb0, rb1, mb0, mb1, rows0, rows1,
            sbacc, racc, macc, semg0, semg1, sema0, sema1):
    c, s, base = _wid_base()
    pltpu.sync_copy(src_hbm.at[pl.ds(base, PAIRS_W)], sall)
    pltpu.sync_copy(dst_hbm.at[pl.ds(base, PAIRS_W)], dall)
    _zero_rows(rows0)
    _zero_vec(rb0, K)
    _zero_vec(rb1, K)
    _zero_vec(mb0, K)
    _zero_vec(mb1, K)

    def zi(i, _):
        scb0[pl.ds(i * 16, 16)] = jnp.zeros((16,), jnp.int32)
        scb1[pl.ds(i * 16, 16)] = jnp.zeros((16,), jnp.int32)
        return 0
    lax.fori_loop(0, K // 16, zi, 0)
    for j in range(TV // K):
        pltpu.sync_copy(rows0, sbacc.at[pl.ds(s * TV + j * K, K)])
        pltpu.sync_copy(rb0, racc.at[pl.ds(s * TV + j * K, K)])
        pltpu.sync_copy(rb0, macc.at[pl.ds(s * TV + j * K, K)])
    plsc.subcore_barrier()
    pltpu.async_copy(rb0, racc.at[scb0], sema0, add=True)
    pltpu.async_copy(mb0, macc.at[scb0], sema0, add=True)
    pltpu.async_copy(rb1, racc.at[scb1], sema1, add=True)
    pltpu.async_copy(mb1, macc.at[scb1], sema1, add=True)

    def stage(j, sb, db, tv, tq, te, tqe, rows, semg):
        _copy_idx(sall, sb, j)
        _copy_idx(dall, db, j)
        pltpu.async_copy(b_hbm.at[db], rows, semg)
        pltpu.async_copy(sv_hbm.at[sb], tv, semg)
        pltpu.async_copy(qv_hbm.at[sb], tq, semg)
        pltpu.async_copy(se_hbm.at[db], te, semg)
        pltpu.async_copy(qe_hbm.at[db], tqe, semg)

    def finish(sb, db, scb, tv, tq, te, tqe, rb, mb, rows, semg, sema):
        pltpu.make_async_copy(rb, racc.at[scb], sema).wait()
        pltpu.make_async_copy(mb, macc.at[scb], sema).wait()
        pltpu.make_async_copy(b_hbm.at[db], rows, semg).wait()
        pltpu.make_async_copy(sv_hbm.at[sb], tv, semg).wait()
        pltpu.make_async_copy(qv_hbm.at[sb], tq, semg).wait()
        pltpu.make_async_copy(se_hbm.at[db], te, semg).wait()
        pltpu.make_async_copy(qe_hbm.at[db], tqe, semg).wait()
        for t in range(K // 16):
            sl = pl.ds(t * 16, 16)
            scb[sl] = sb[sl]
            m = tv[sl] + te[sl]
            w = tq[sl] + tqe[sl] - m * m + EPS
            r = _rsqrt16(w)
            rb[sl] = r
            mb[sl] = m * r
        pltpu.async_copy(rb, racc.at[scb], sema, add=True)
        pltpu.async_copy(mb, macc.at[scb], sema, add=True)

        def rowfn(jj, _):
            j4 = jj * 4
            for u in range(4):
                rj = plsc.load_gather(rb, [jnp.full((16,), j4 + u, jnp.int32)])
                for k in range(8):
                    sl = pl.ds(k * 16, 16)
                    rows[j4 + u, sl] = rows[j4 + u, sl] * rj
            return 0
        lax.fori_loop(0, K // 4, rowfn, 0)
        pltpu.sync_copy(rows, sbacc.at[sb], add=True)

    def fin0():
        finish(s0, d0, scb0, tv0, tq0, te0, tqe0, rb0, mb0, rows0,
               semg0, sema0)

    def fin1():
        finish(s1, d1, scb1, tv1, tq1, te1, tqe1, rb1, mb1, rows1,
               semg1, sema1)

    stage(0, s0, d0, tv0, tq0, te0, tqe0, rows0, semg0)

    def body(i, _):
        stage(2 * i + 1, s1, d1, tv1, tq1, te1, tqe1, rows1, semg1)
        fin0()
        stage(2 * i + 2, s0, d0, tv0, tq0, te0, tqe0, rows0, semg0)
        fin1()
        return 0
    lax.fori_loop(0, (NCHUNK - 1) // 2, body, 0)
    fin0()
    pltpu.make_async_copy(rb0, racc.at[scb0], sema0).wait()
    pltpu.make_async_copy(mb0, macc.at[scb0], sema0).wait()
    pltpu.make_async_copy(rb1, racc.at[scb1], sema1).wait()
    pltpu.make_async_copy(mb1, macc.at[scb1], sema1).wait()
    plsc.subcore_barrier()
    pltpu.sync_copy(sbacc.at[pl.ds(s * TV, TV)], sb_out.at[c, pl.ds(s * TV, TV)])

    @pl.when(s == 0)
    def _():
        pltpu.sync_copy(racc, r_out.at[pl.ds(c * NP, NP)])
        pltpu.sync_copy(macc, mr_out.at[pl.ds(c * NP, NP)])


_IVEC = pltpu.VMEM((PAIRS_W,), jnp.int32)
_IK = pltpu.VMEM((K,), jnp.int32)
_FK = pltpu.VMEM((K,), jnp.float32)
_ROWS = pltpu.VMEM((K, C), jnp.float32)

_counts_call = functools.partial(
    pl.kernel, _sc_counts, mesh=_SC_MESH, compiler_params=_SC_PARAMS,
    out_type=[jax.ShapeDtypeStruct((NC * NP,), jnp.float32),
              jax.ShapeDtypeStruct((NC * EP,), jnp.float32)],
    scratch_types=[_IVEC, _IVEC, _IK, _IK, _IK, _IK, _FK,
                   pltpu.VMEM((TV,), jnp.float32),
                   pltpu.VMEM_SHARED((NP,), jnp.float32),
                   pltpu.VMEM_SHARED((EP,), jnp.float32),
                   pltpu.SemaphoreType.DMA, pltpu.SemaphoreType.DMA])

_v2e_call = functools.partial(
    pl.kernel, _sc_v2e, mesh=_SC_MESH, compiler_params=_SC_PARAMS,
    out_type=[jax.ShapeDtypeStruct((NC, EP, C), jnp.float32)],
    scratch_types=[_IVEC, _IVEC, _IK, _IK, _IK, _IK, _ROWS, _ROWS,
                   pltpu.VMEM_SHARED((EP, C), jnp.float32),
                   pltpu.SemaphoreType.DMA, pltpu.SemaphoreType.DMA,
                   pltpu.SemaphoreType.DMA, pltpu.SemaphoreType.DMA])

_e2v_call = functools.partial(
    pl.kernel, _sc_e2v, mesh=_SC_MESH, compiler_params=_SC_PARAMS,
    out_type=[jax.ShapeDtypeStruct((NC, NP, C), jnp.float32),
              jax.ShapeDtypeStruct((NC * NP,), jnp.float32),
              jax.ShapeDtypeStruct((NC * NP,), jnp.float32)],
    scratch_types=[_IVEC, _IVEC, _IK, _IK, _IK, _IK, _IK, _IK,
                   _FK, _FK, _FK, _FK, _FK, _FK, _FK, _FK,
                   _FK, _FK, _FK, _FK, _ROWS, _ROWS,
                   pltpu.VMEM_SHARED((NP, C), jnp.float32),
                   pltpu.VMEM_SHARED((NP,), jnp.float32),
                   pltpu.VMEM_SHARED((NP,), jnp.float32),
                   pltpu.SemaphoreType.DMA, pltpu.SemaphoreType.DMA,
                   pltpu.SemaphoreType.DMA, pltpu.SemaphoreType.DMA])


def _row_spec(blk):
    return pl.BlockSpec((blk, C), lambda i: (i, 0))


def _full(shape):
    return pl.BlockSpec(shape, lambda i: tuple(0 for _ in shape))


def _pad1(x, n):
    return jnp.pad(jnp.reshape(x, (-1,)), (0, n - x.shape[0]))


def kernel(X, src, dst, all_batch, lin_in_w, lin_in_b, w1_ln0_g, w1_ln0_b,
           w1_lin0_w, w1_lin0_b, w1_ln1_g, w1_ln1_b, w1_lin1_w, w1_lin1_b,
           w2_ln_g, w2_ln_b, w2_lin_w, w2_lin_b, w3_ln_g, w3_ln_b, w3_lin_w,
           w3_lin_b, clf_lin0_w, clf_lin0_b, clf_ln_g, clf_ln_b, clf_lin1_w,
           clf_lin1_b):
    f32 = jnp.float32
    row = lambda v: jnp.reshape(v, (1, -1))
    col1 = lambda i: (i, 0)
    _sv_spec = pl.BlockSpec((ROWB, 1), col1)
    _part_row = pl.BlockSpec((NC, ROWB, C), lambda i: (0, i, 0))
    _part_sv = pl.BlockSpec((NC, ROWB, 1), lambda i: (0, i, 0))

    gb = jnp.stack([w2_ln_g, w2_ln_b], axis=0)
    gwc = pl.pallas_call(
        _k_prep,
        grid=(1,),
        in_specs=[_full((2, 2 * C)), _full((2 * C, C)), _full((1, C))],
        out_specs=_full((2, C)),
        out_shape=jax.ShapeDtypeStruct((2, C), f32),
    )(gb, w2_lin_w, row(w2_lin_b))

    cv_p, ce_p = _counts_call()(src, dst)
    cv3 = jnp.reshape(cv_p, (NC, NP))[:, :N, None]
    ce3 = jnp.reshape(ce_p, (NC, EP))[:, :EH, None]

    w2top = w2_lin_w[:C]
    w2bot = w2_lin_w[C:]
    g2lo = row(w2_ln_g[:C])
    g2hi = row(w2_ln_g[C:])
    w1_args = (row(w1_ln0_g), row(w1_ln0_b), w1_lin0_w, row(w1_lin0_b),
               row(w1_ln1_g), row(w1_ln1_b), w1_lin1_w, row(w1_lin1_b),
               g2lo, w2top)
    w1_specs = ([_full((1, C))] * 2 + [_full((C, C)), _full((1, C))]
                + [_full((1, C))] * 2 + [_full((C, C)), _full((1, C))]
                + [_full((1, C)), _full((C, C))])
    w3_args = (row(w3_ln_g), row(w3_ln_b), w3_lin_w, row(w3_lin_b))
    w3_specs = [_full((1, C)), _full((1, C)), _full((C, C)), _full((1, C))]
    la_out_specs = [_row_spec(ROWB), _row_spec(ROWB), _sv_spec, _sv_spec]
    la_out_shape = [jax.ShapeDtypeStruct((N, C), f32),
                    jax.ShapeDtypeStruct((N, C), f32),
                    jax.ShapeDtypeStruct((N, 1), f32),
                    jax.ShapeDtypeStruct((N, 1), f32)]

    x0, h2, a_mat, sv, qv = pl.pallas_call(
        _k_in_a,
        grid=(GRID,),
        in_specs=[_row_spec(ROWB), _full((C, C)), _full((1, C))] + w1_specs,
        out_specs=[_row_spec(ROWB)] + la_out_specs,
        out_shape=[jax.ShapeDtypeStruct((N, C), f32)] + la_out_shape,
    )(X, lin_in_w, row(lin_in_b), *w1_args)

    def sparse_phase(h2, sv, qv):
        (xe_p,) = _v2e_call()(h2, src, dst)
        b_mat, se, qe = pl.pallas_call(
            _k_edge,
            grid=(EGRID,),
            in_specs=[pl.BlockSpec((NC, EROWB, C), lambda i: (0, i, 0)),
                      pl.BlockSpec((NC, EROWB, 1), lambda i: (0, i, 0)),
                      _full((1, C)), _full((C, C))],
            out_specs=[_row_spec(EROWB),
                       pl.BlockSpec((EROWB, 1), col1),
                       pl.BlockSpec((EROWB, 1), col1)],
            out_shape=[jax.ShapeDtypeStruct((EH, C), f32),
                       jax.ShapeDtypeStruct((EH, 1), f32),
                       jax.ShapeDtypeStruct((EH, 1), f32)],
        )(xe_p[:, :EH], ce3, g2hi, w2bot)
        sb_p, r_p, mr_p = _e2v_call()(
            b_mat, src, dst, _pad1(sv, NP), _pad1(qv, NP),
            _pad1(se, EP), _pad1(qe, EP))
        return (sb_p[:, :N], jnp.reshape(r_p, (NC, NP))[:, :N, None],
                jnp.reshape(mr_p, (NC, NP))[:, :N, None])

    sb1, r1, mr1 = sparse_phase(h2, sv, qv)

    h2, a_mat, sv, qv = pl.pallas_call(
        _k_vertex_a,
        grid=(GRID,),
        in_specs=[_part_row, _part_sv, _part_sv, _part_sv,
                  _row_spec(ROWB), _row_spec(ROWB), _full((2, C))]
        + w3_specs + w1_specs,
        out_specs=la_out_specs,
        out_shape=la_out_shape,
    )(sb1, r1, mr1, cv3, a_mat, x0, gwc, *w3_args, *w1_args)

    sb2, r2, mr2 = sparse_phase(h2, sv, qv)

    wc1p = jnp.pad(clf_lin1_w, ((0, 0), (0, C - NCLS)))
    bc1p = row(jnp.pad(clf_lin1_b, (0, C - NCLS)))
    ab3 = jnp.reshape(all_batch.astype(jnp.int32), (GRID, 1, ROWB))
    readout = pl.pallas_call(
        _k_vertex_clf,
        grid=(GRID,),
        in_specs=[_part_row, _part_sv, _part_sv, _part_sv,
                  _row_spec(ROWB), _row_spec(ROWB), _full((2, C))]
        + w3_specs
        + [_full((C, C)), _full((1, C)), _full((1, C)), _full((1, C)),
           _full((C, C)), _full((1, C)),
           pl.BlockSpec((1, 1, ROWB), lambda i: (i, 0, 0))],
        out_specs=_full((NGRAPH, NCLS)),
        out_shape=jax.ShapeDtypeStruct((NGRAPH, NCLS), f32),
        scratch_shapes=[pltpu.VMEM((NGRAPH, C), f32),
                        pltpu.VMEM((NGRAPH, C), f32)],
    )(sb2, r2, mr2, cv3, a_mat, x0, gwc, *w3_args,
      clf_lin0_w, row(clf_lin0_b), row(clf_ln_g), row(clf_ln_b),
      wc1p, bc1p, ab3)
    return readout
